# Initial kernel scaffold; baseline (speedup 1.0000x reference)
import jax
import jax.numpy as jnp
from jax.experimental import pallas as pl


def kernel(features, rel_emb, adj, r_index, r_val, high_nei, attn_k0, attn_k1, high_att0, high_att1):
    raise NotImplementedError("write your pallas kernel here")


# speedup vs baseline: 1.0084x; 1.0084x over previous
import jax
import jax.numpy as jnp
from jax.experimental import pallas as pl

NODE_SIZE = 10000
TRIPLE_SIZE = 320000
DEPTH = 2


def _tanh_kernel(x_ref, o_ref):
    o_ref[...] = jnp.tanh(x_ref[...])


def _ptanh(x):
    return pl.pallas_call(
        _tanh_kernel,
        out_shape=jax.ShapeDtypeStruct(x.shape, x.dtype),
    )(x)


def _segment_softmax(vals, seg, num_segments):
    m = jax.ops.segment_max(vals, seg, num_segments=num_segments)
    m = jnp.where(jnp.isfinite(m), m, 0.0)
    e = jnp.exp(vals - m[seg])
    s = jax.ops.segment_sum(e, seg, num_segments=num_segments)
    return e / (s[seg] + 1e-16)


def kernel(features, rel_emb, adj, r_index, r_val, high_nei,
           attn_k0, attn_k1, high_att0, high_att1):
    attn_kernels = [attn_k0, attn_k1]
    high_atts = [high_att0, high_att1]
    outputs = []
    feats = _ptanh(features)
    outputs.append(feats)
    tri_rel = jax.ops.segment_sum(r_val[:, None] * rel_emb[r_index[1]],
                                  r_index[0], num_segments=TRIPLE_SIZE)
    norm = jnp.sqrt(jnp.sum(tri_rel * tri_rel, axis=1, keepdims=True))
    tri_rel = tri_rel / jnp.maximum(norm, 1e-12)
    for l in range(DEPTH):
        neighs = feats[adj[1]]
        neighs = neighs - 2.0 * jnp.sum(neighs * tri_rel, axis=1, keepdims=True) * tri_rel
        att = jnp.squeeze(tri_rel @ attn_kernels[l], axis=-1)
        att = _segment_softmax(att, adj[0], NODE_SIZE)
        new_feats = jax.ops.segment_sum(neighs * att[:, None], adj[0], num_segments=NODE_SIZE)
        feats = _ptanh(new_feats)
        outputs.append(feats)
    for l in range(DEPTH):
        neighs = feats[high_nei[1]]
        att = jnp.squeeze(neighs @ high_atts[l], axis=-1)
        att = _segment_softmax(att, high_nei[0], NODE_SIZE)
        new_feats = jax.ops.segment_sum(neighs * att[:, None], high_nei[0], num_segments=NODE_SIZE)
        feats = _ptanh(new_feats)
        outputs.append(feats)
    return jnp.concatenate(outputs, axis=-1)


# speedup vs baseline: 4.0064x; 3.9732x over previous
import jax
import jax.numpy as jnp
from jax import lax
from jax.experimental import pallas as pl
from jax.experimental.pallas import tpu as pltpu
from jax.experimental.pallas import tpu_sc as plsc

N_NODE = 10000
N_DIM = 128
N_EDGE = 320000
N_TRI = 320000
LANES = 16
DSL = N_DIM // LANES

NC, NS = 2, 16
NW = NC * NS

TB = 128
PAD = 512
SEG_CHUNK = 512
N_SEG_CHUNKS = N_TRI // SEG_CHUNK
SEG_CHUNKS_PER_W = -(-N_SEG_CHUNKS // NW)
NODE_CHUNK = 312
NODE_LAST = N_NODE - (NW - 1) * NODE_CHUNK
NODE_ACC = NODE_LAST

_mesh = plsc.VectorSubcoreMesh(
    core_axis_name="c", subcore_axis_name="s", num_cores=NC, num_subcores=NS)


def _worker_id():
    return lax.axis_index("s") * NC + lax.axis_index("c")


def _lane_sum(x):
    parts = [x[t] for t in range(LANES)]
    while len(parts) > 1:
        parts = [parts[i] + parts[i + 1] for i in range(0, len(parts) - 1, 2)] + (
            [parts[-1]] if len(parts) % 2 else [])
    return parts[0]


def _zero_rows(acc_v, nrows):
    z = jnp.zeros((LANES,), jnp.float32)

    def zrow(r, carry):
        for c in range(DSL):
            acc_v[r, pl.ds(c * LANES, LANES)] = z
        return carry
    lax.fori_loop(0, nrows, zrow, None)


def _s1_body(bounds, relids, segids, rval, rel_emb, tri_out,
             bnd_v, idx_v, seg_v, val_v, rows_v, acc_v, sem):
    w = _worker_id()

    def chunk(ci, carry):
        j = w + ci * NW

        @pl.when(j < N_SEG_CHUNKS)
        def _():
            pltpu.sync_copy(bounds.at[j], bnd_v)
            bv = bnd_v[...]
            t0 = bv[0]
            t1 = bv[1]
            seg_base = j * SEG_CHUNK
            _zero_rows(acc_v, SEG_CHUNK)
            a8 = (t0 // 8) * 8
            nbat = (t1 - a8 + TB - 1) // TB

            def batch(k, bcarry):
                base = a8 + k * TB
                pltpu.sync_copy(relids.at[pl.ds(base, TB)], idx_v)
                pltpu.sync_copy(segids.at[pl.ds(base, TB)], seg_v.at[pl.ds(0, TB)])
                pltpu.sync_copy(rval.at[pl.ds(base, TB)], val_v.at[pl.ds(0, TB)])
                pltpu.async_copy(rel_emb.at[idx_v], rows_v, sem).wait()
                lo = jnp.maximum(t0 - base, 0)
                hi = jnp.minimum(t1 - base, TB)

                def tri(i, tcarry):
                    s = seg_v[pl.ds(i, LANES)][0] - seg_base
                    vv = val_v[pl.ds(i, LANES)][0]
                    for c in range(DSL):
                        sl = pl.ds(c * LANES, LANES)
                        acc_v[s, sl] = acc_v[s, sl] + vv * rows_v[i, sl]
                    return tcarry
                lax.fori_loop(lo, hi, tri, None)
                return bcarry
            lax.fori_loop(0, nbat, batch, None)
            pltpu.sync_copy(acc_v, tri_out.at[pl.ds(seg_base, SEG_CHUNK)])
        return carry
    lax.fori_loop(0, SEG_CHUNKS_PER_W, chunk, None)


def _s1_call(bounds, relids, segids, rval, rel_emb):
    f = pl.kernel(
        _s1_body,
        out_type=jax.ShapeDtypeStruct((N_TRI + PAD, N_DIM), jnp.float32),
        mesh=_mesh,
        scratch_types=[
            pltpu.VMEM((LANES,), jnp.int32),
            pltpu.VMEM((TB,), jnp.int32),
            pltpu.VMEM((TB + LANES,), jnp.int32),
            pltpu.VMEM((TB + LANES,), jnp.float32),
            pltpu.VMEM((TB, N_DIM), jnp.float32),
            pltpu.VMEM((SEG_CHUNK, N_DIM), jnp.float32),
            pltpu.SemaphoreType.DMA,
        ],
    )
    return f(bounds, relids, segids, rval, rel_emb)


def _s3_body(bounds, adj0, adj1, ev, feats, vhat, out,
             bnd_v, seg_v, idx_v, e_v, f_rows, v_rows, acc_v, s_sm, sem):
    w = _worker_id()
    pltpu.sync_copy(bounds.at[w], bnd_v)
    bv = bnd_v[...]
    b0 = bv[0]
    b1 = bv[1]
    n0 = w * NODE_CHUNK
    _zero_rows(acc_v, NODE_ACC)

    def zs(r, carry):
        s_sm[r] = 0.0
        return carry
    lax.fori_loop(0, NODE_ACC, zs, None)

    a8 = (b0 // 8) * 8
    nbat = (b1 - a8 + TB - 1) // TB

    def batch(k, bcarry):
        base = a8 + k * TB
        pltpu.sync_copy(adj0.at[pl.ds(base, TB)], seg_v.at[pl.ds(0, TB)])
        pltpu.sync_copy(adj1.at[pl.ds(base, TB)], idx_v)
        pltpu.sync_copy(ev.at[pl.ds(base, TB)], e_v.at[pl.ds(0, TB)])
        pltpu.sync_copy(vhat.at[pl.ds(base, TB)], v_rows)
        pltpu.async_copy(feats.at[idx_v], f_rows, sem).wait()
        lo = jnp.maximum(b0 - base, 0)
        hi = jnp.minimum(b1 - base, TB)

        def edge(i, ecarry):
            d = seg_v[pl.ds(i, LANES)][0] - n0
            ee = e_v[pl.ds(i, LANES)][0]
            s_sm[d] = s_sm[d] + ee
            pacc = f_rows[i, pl.ds(0, LANES)] * v_rows[i, pl.ds(0, LANES)]
            for c in range(1, DSL):
                sl = pl.ds(c * LANES, LANES)
                pacc = pacc + f_rows[i, sl] * v_rows[i, sl]
            dot = _lane_sum(pacc)
            coef = -2.0 * dot * ee
            for c in range(DSL):
                sl = pl.ds(c * LANES, LANES)
                acc_v[d, sl] = acc_v[d, sl] + ee * f_rows[i, sl] + coef * v_rows[i, sl]
            return ecarry
        lax.fori_loop(lo, hi, edge, None)
        return bcarry
    lax.fori_loop(0, nbat, batch, None)

    def fin(r, carry):
        rsv = 1.0 / jnp.full((LANES,), s_sm[r] + 1e-16, jnp.float32)
        for c in range(DSL):
            sl = pl.ds(c * LANES, LANES)
            acc_v[r, sl] = acc_v[r, sl] * rsv
        return carry
    lax.fori_loop(0, NODE_ACC, fin, None)

    pltpu.sync_copy(acc_v.at[pl.ds(0, NODE_CHUNK)], out.at[pl.ds(n0, NODE_CHUNK)])

    @pl.when(w == NW - 1)
    def _():
        pltpu.sync_copy(
            acc_v.at[pl.ds(NODE_CHUNK, NODE_LAST - NODE_CHUNK)],
            out.at[pl.ds(NW * NODE_CHUNK, NODE_LAST - NODE_CHUNK)])


def _s3_call(bounds, adj0, adj1, ev, feats, vhat):
    f = pl.kernel(
        _s3_body,
        out_type=jax.ShapeDtypeStruct((N_NODE, N_DIM), jnp.float32),
        mesh=_mesh,
        scratch_types=[
            pltpu.VMEM((LANES,), jnp.int32),
            pltpu.VMEM((TB + LANES,), jnp.int32),
            pltpu.VMEM((TB,), jnp.int32),
            pltpu.VMEM((TB + LANES,), jnp.float32),
            pltpu.VMEM((TB, N_DIM), jnp.float32),
            pltpu.VMEM((TB, N_DIM), jnp.float32),
            pltpu.VMEM((NODE_ACC, N_DIM), jnp.float32),
            pltpu.SMEM((NODE_ACC,), jnp.float32),
            pltpu.SemaphoreType.DMA,
        ],
    )
    return f(bounds, adj0, adj1, ev, feats, vhat)


def _s4_body(bounds, hi0, hi1, pe, feats, out,
             bnd_v, seg_v, src_v, idx_v, pe_v, f_rows, acc_v, s_sm, sem):
    w = _worker_id()
    pltpu.sync_copy(bounds.at[w], bnd_v)
    bv = bnd_v[...]
    b0 = bv[0]
    b1 = bv[1]
    n0 = w * NODE_CHUNK
    pltpu.sync_copy(pe, pe_v.at[pl.ds(0, N_NODE)])
    _zero_rows(acc_v, NODE_ACC)

    def zs(r, carry):
        s_sm[r] = 0.0
        return carry
    lax.fori_loop(0, NODE_ACC, zs, None)

    a8 = (b0 // 8) * 8
    nbat = (b1 - a8 + TB - 1) // TB

    def batch(k, bcarry):
        base = a8 + k * TB
        pltpu.sync_copy(hi0.at[pl.ds(base, TB)], seg_v.at[pl.ds(0, TB)])
        pltpu.sync_copy(hi1.at[pl.ds(base, TB)], idx_v)
        pltpu.sync_copy(hi1.at[pl.ds(base, TB)], src_v.at[pl.ds(0, TB)])
        pltpu.async_copy(feats.at[idx_v], f_rows, sem).wait()
        lo = jnp.maximum(b0 - base, 0)
        hi = jnp.minimum(b1 - base, TB)

        def edge(i, ecarry):
            d = seg_v[pl.ds(i, LANES)][0] - n0
            sidx = src_v[pl.ds(i, LANES)][0]
            q = pe_v[pl.ds(sidx, LANES)][0]
            s_sm[d] = s_sm[d] + q
            for c in range(DSL):
                sl = pl.ds(c * LANES, LANES)
                acc_v[d, sl] = acc_v[d, sl] + q * f_rows[i, sl]
            return ecarry
        lax.fori_loop(lo, hi, edge, None)
        return bcarry
    lax.fori_loop(0, nbat, batch, None)

    def fin(r, carry):
        rsv = 1.0 / jnp.full((LANES,), s_sm[r] + 1e-16, jnp.float32)
        for c in range(DSL):
            sl = pl.ds(c * LANES, LANES)
            acc_v[r, sl] = acc_v[r, sl] * rsv
        return carry
    lax.fori_loop(0, NODE_ACC, fin, None)

    pltpu.sync_copy(acc_v.at[pl.ds(0, NODE_CHUNK)], out.at[pl.ds(n0, NODE_CHUNK)])

    @pl.when(w == NW - 1)
    def _():
        pltpu.sync_copy(
            acc_v.at[pl.ds(NODE_CHUNK, NODE_LAST - NODE_CHUNK)],
            out.at[pl.ds(NW * NODE_CHUNK, NODE_LAST - NODE_CHUNK)])


def _s4_call(bounds, hi0, hi1, pe, feats):
    f = pl.kernel(
        _s4_body,
        out_type=jax.ShapeDtypeStruct((N_NODE, N_DIM), jnp.float32),
        mesh=_mesh,
        scratch_types=[
            pltpu.VMEM((LANES,), jnp.int32),
            pltpu.VMEM((TB + LANES,), jnp.int32),
            pltpu.VMEM((TB + LANES,), jnp.int32),
            pltpu.VMEM((TB,), jnp.int32),
            pltpu.VMEM((N_NODE + LANES,), jnp.float32),
            pltpu.VMEM((TB, N_DIM), jnp.float32),
            pltpu.VMEM((NODE_ACC, N_DIM), jnp.float32),
            pltpu.SMEM((NODE_ACC,), jnp.float32),
            pltpu.SemaphoreType.DMA,
        ],
    )
    return f(bounds, hi0, hi1, pe, feats)


def _tanh_body(x_ref, o_ref):
    o_ref[...] = jnp.tanh(x_ref[...])


def _tanh_call(x):
    return pl.pallas_call(
        _tanh_body,
        out_shape=jax.ShapeDtypeStruct(x.shape, jnp.float32),
    )(x)


def _t1_body(tri_ref, k01_ref, v_ref, e_ref):
    x = tri_ref[...]
    ss = jnp.sum(x * x, axis=1, keepdims=True)
    v = x / jnp.maximum(jnp.sqrt(ss), 1e-12)
    v_ref[...] = v
    e_ref[...] = jnp.exp(v @ k01_ref[...])


def _t1_call(tri, k01):
    n = tri.shape[0]
    br = 1024
    return pl.pallas_call(
        _t1_body,
        grid=(n // br,),
        in_specs=[pl.BlockSpec((br, N_DIM), lambda i: (i, 0)),
                  pl.BlockSpec((N_DIM, 2), lambda i: (0, 0))],
        out_specs=[pl.BlockSpec((br, N_DIM), lambda i: (i, 0)),
                   pl.BlockSpec((br, 2), lambda i: (i, 0))],
        out_shape=[jax.ShapeDtypeStruct((n, N_DIM), jnp.float32),
                   jax.ShapeDtypeStruct((n, 2), jnp.float32)],
    )(tri, k01)


def _t2pe_body(raw_ref, ha_ref, f_ref, pe_ref):
    f = jnp.tanh(raw_ref[...])
    f_ref[...] = f
    p = jnp.exp(f @ ha_ref[...])
    pe_ref[...] = p


def _t2pe_call(raw, ha):
    return pl.pallas_call(
        _t2pe_body,
        out_shape=[jax.ShapeDtypeStruct((N_NODE, N_DIM), jnp.float32),
                   jax.ShapeDtypeStruct((N_NODE, 1), jnp.float32)],
    )(raw, ha)


def kernel(features, rel_emb, adj, r_index, r_val, high_nei,
           attn_k0, attn_k1, high_att0, high_att1):
    r0, r1 = r_index[0], r_index[1]
    a0, a1 = adj[0], adj[1]
    h0, h1 = high_nei[0], high_nei[1]

    seg_grid = jnp.arange(N_SEG_CHUNKS + 1, dtype=jnp.int32) * SEG_CHUNK
    sb = jnp.searchsorted(r0, seg_grid, side="left").astype(jnp.int32)
    s1_bounds = jnp.zeros((NW * SEG_CHUNKS_PER_W, LANES), jnp.int32)
    s1_bounds = s1_bounds.at[:N_SEG_CHUNKS, 0].set(sb[:-1])
    s1_bounds = s1_bounds.at[:N_SEG_CHUNKS, 1].set(sb[1:])

    node_grid = jnp.concatenate([
        jnp.arange(NW, dtype=jnp.int32) * NODE_CHUNK,
        jnp.array([N_NODE], jnp.int32)])
    ab = jnp.searchsorted(a0, node_grid, side="left").astype(jnp.int32)
    s3_bounds = jnp.zeros((NW, LANES), jnp.int32)
    s3_bounds = s3_bounds.at[:, 0].set(ab[:-1]).at[:, 1].set(ab[1:])
    hb = jnp.searchsorted(h0, node_grid, side="left").astype(jnp.int32)
    s4_bounds = jnp.zeros((NW, LANES), jnp.int32)
    s4_bounds = s4_bounds.at[:, 0].set(hb[:-1]).at[:, 1].set(hb[1:])

    pad1 = lambda x: jnp.pad(x, (0, PAD))
    r0p, r1p, rvp = pad1(r0), pad1(r1), pad1(r_val)
    a0p, a1p = pad1(a0), pad1(a1)
    h0p, h1p = pad1(h0), pad1(h1)

    tri = _s1_call(s1_bounds, r1p, r0p, rvp, rel_emb)
    k01 = jnp.concatenate([attn_k0, attn_k1], axis=1)
    vhat, e01 = _t1_call(tri, k01)
    e0 = jnp.asarray(e01[:, 0])
    e1 = jnp.asarray(e01[:, 1])

    f0 = _tanh_call(features)
    raw1 = _s3_call(s3_bounds, a0p, a1p, e0, f0, vhat)
    f1 = _tanh_call(raw1)
    raw2 = _s3_call(s3_bounds, a0p, a1p, e1, f1, vhat)
    f2, pe0 = _t2pe_call(raw2, high_att0)
    raw3 = _s4_call(s4_bounds, h0p, h1p, jnp.squeeze(pe0, -1), f2)
    f3, pe1 = _t2pe_call(raw3, high_att1)
    raw4 = _s4_call(s4_bounds, h0p, h1p, jnp.squeeze(pe1, -1), f3)
    f4 = _tanh_call(raw4)
    return jnp.concatenate([f0, f1, f2, f3, f4], axis=-1)


# speedup vs baseline: 4.0714x; 1.0162x over previous
import jax
import jax.numpy as jnp
from jax import lax
from jax.experimental import pallas as pl
from jax.experimental.pallas import tpu as pltpu
from jax.experimental.pallas import tpu_sc as plsc

N_NODE = 10000
N_DIM = 128
N_EDGE = 320000
N_TRI = 320000
LANES = 16
DSL = N_DIM // LANES

NC, NS = 2, 16
NW = NC * NS

TB = 128
PAD = 512
SEG_CHUNK = 512
N_SEG_CHUNKS = N_TRI // SEG_CHUNK
SEG_CHUNKS_PER_W = -(-N_SEG_CHUNKS // NW)
NODE_CHUNK = 312
NODE_LAST = N_NODE - (NW - 1) * NODE_CHUNK
NODE_ACC = NODE_LAST

_mesh = plsc.VectorSubcoreMesh(
    core_axis_name="c", subcore_axis_name="s", num_cores=NC, num_subcores=NS)


def _worker_id():
    return lax.axis_index("s") * NC + lax.axis_index("c")


def _zero_rows(acc_v, nrows):
    z = jnp.zeros((LANES,), jnp.float32)

    def zrow(r, carry):
        for c in range(DSL):
            acc_v[r, pl.ds(c * LANES, LANES)] = z
        return carry
    lax.fori_loop(0, nrows, zrow, None)


def _zero_svec(s_v, n):
    z = jnp.zeros((LANES,), jnp.float32)

    def zs(r, carry):
        s_v[r, pl.ds(0, LANES)] = z
        return carry
    lax.fori_loop(0, n, zs, None)


def _s1_body(bounds, relids, segids, rval, rel_emb, tri_out,
             bnd_v, idx_v, seg_v, val_v, rows_v, acc_v, sem, sem2, sem3, sem4):
    w = _worker_id()

    def chunk(ci, carry):
        j = w + ci * NW

        @pl.when(j < N_SEG_CHUNKS)
        def _():
            pltpu.sync_copy(bounds.at[j], bnd_v)
            bv = bnd_v[...]
            t0 = bv[0]
            t1 = bv[1]
            seg_base = j * SEG_CHUNK
            _zero_rows(acc_v, SEG_CHUNK)
            a8 = (t0 // 8) * 8
            nbat = (t1 - a8 + TB - 1) // TB

            def batch(k, bcarry):
                base = a8 + k * TB
                c1 = pltpu.async_copy(relids.at[pl.ds(base, TB)], idx_v, sem2)
                c2 = pltpu.async_copy(
                    segids.at[pl.ds(base, TB)], seg_v.at[pl.ds(0, TB)], sem3)
                c3 = pltpu.async_copy(rval.at[pl.ds(base, TB)], val_v, sem4)
                c1.wait()
                cg = pltpu.async_copy(rel_emb.at[idx_v], rows_v, sem)
                c2.wait()
                c3.wait()
                cg.wait()
                lo = jnp.maximum(t0 - base, 0)
                hi = jnp.minimum(t1 - base, TB)

                def tri(i, tcarry):
                    s = seg_v[pl.ds(i, LANES)][0] - seg_base
                    ibc = jnp.full((LANES,), i, jnp.int32)
                    vvb = plsc.load_gather(val_v, [ibc])
                    for c in range(DSL):
                        sl = pl.ds(c * LANES, LANES)
                        acc_v[s, sl] = acc_v[s, sl] + vvb * rows_v[i, sl]
                    return tcarry
                lax.fori_loop(lo, hi, tri, None)
                return bcarry
            lax.fori_loop(0, nbat, batch, None)
            pltpu.sync_copy(acc_v, tri_out.at[pl.ds(seg_base, SEG_CHUNK)])
        return carry
    lax.fori_loop(0, SEG_CHUNKS_PER_W, chunk, None)


def _s1_call(bounds, relids, segids, rval, rel_emb):
    f = pl.kernel(
        _s1_body,
        out_type=jax.ShapeDtypeStruct((N_TRI + PAD, N_DIM), jnp.float32),
        mesh=_mesh,
        compiler_params=pltpu.CompilerParams(needs_layout_passes=False),
        scratch_types=[
            pltpu.VMEM((LANES,), jnp.int32),
            pltpu.VMEM((TB,), jnp.int32),
            pltpu.VMEM((TB + LANES,), jnp.int32),
            pltpu.VMEM((TB,), jnp.float32),
            pltpu.VMEM((TB, N_DIM), jnp.float32),
            pltpu.VMEM((SEG_CHUNK, N_DIM), jnp.float32),
            pltpu.SemaphoreType.DMA,
            pltpu.SemaphoreType.DMA,
            pltpu.SemaphoreType.DMA,
            pltpu.SemaphoreType.DMA,
        ],
    )
    return f(bounds, relids, segids, rval, rel_emb)


def _s3_body(bounds, adj0, adj1, ev, feats, vhat, out,
             bnd_v, seg_v, e_v, idx_v, f_rows, v_rows, dot_scr, acc_v, s_v,
             sem, sem2, sem3, sem4, sem5):
    w = _worker_id()
    pltpu.sync_copy(bounds.at[w], bnd_v)
    bv = bnd_v[...]
    b0 = bv[0]
    b1 = bv[1]
    n0 = w * NODE_CHUNK
    _zero_rows(acc_v, NODE_ACC)
    _zero_svec(s_v, NODE_ACC)
    io = lax.iota(jnp.int32, LANES)

    a8 = (b0 // 8) * 8
    nbat = (b1 - a8 + TB - 1) // TB

    def batch(k, bcarry):
        base = a8 + k * TB
        c1 = pltpu.async_copy(adj1.at[pl.ds(base, TB)], idx_v, sem2)
        c2 = pltpu.async_copy(
            adj0.at[pl.ds(base, TB)], seg_v.at[pl.ds(0, TB)], sem3)
        c3 = pltpu.async_copy(ev.at[pl.ds(base, TB)], e_v, sem4)
        c4 = pltpu.async_copy(vhat.at[pl.ds(base, TB)], v_rows, sem5)
        c1.wait()
        cg = pltpu.async_copy(feats.at[idx_v], f_rows, sem)
        c2.wait()
        c3.wait()
        c4.wait()
        cg.wait()
        lo = jnp.maximum(b0 - base, 0)
        hi = jnp.minimum(b1 - base, TB)

        def edge(i, ecarry):
            d = seg_v[pl.ds(i, LANES)][0] - n0
            ibc = jnp.full((LANES,), i, jnp.int32)
            eeb = plsc.load_gather(e_v, [ibc])
            s_v[d, pl.ds(0, LANES)] = s_v[d, pl.ds(0, LANES)] + eeb
            pacc = f_rows[i, pl.ds(0, LANES)] * v_rows[i, pl.ds(0, LANES)]
            for c in range(1, DSL):
                sl = pl.ds(c * LANES, LANES)
                pacc = pacc + f_rows[i, sl] * v_rows[i, sl]
            x = pacc
            for kk in (8, 4, 2, 1):
                dot_scr[...] = x
                x = x + plsc.load_gather(dot_scr, [io ^ kk])
            coefb = x * (-2.0 * eeb)
            for c in range(DSL):
                sl = pl.ds(c * LANES, LANES)
                acc_v[d, sl] = acc_v[d, sl] + eeb * f_rows[i, sl] + coefb * v_rows[i, sl]
            return ecarry
        lax.fori_loop(lo, hi, edge, None)
        return bcarry
    lax.fori_loop(0, nbat, batch, None)

    def fin(r, carry):
        rsv = 1.0 / (s_v[r, pl.ds(0, LANES)] + 1e-16)
        for c in range(DSL):
            sl = pl.ds(c * LANES, LANES)
            acc_v[r, sl] = acc_v[r, sl] * rsv
        return carry
    lax.fori_loop(0, NODE_ACC, fin, None)

    pltpu.sync_copy(acc_v.at[pl.ds(0, NODE_CHUNK)], out.at[pl.ds(n0, NODE_CHUNK)])

    @pl.when(w == NW - 1)
    def _():
        pltpu.sync_copy(
            acc_v.at[pl.ds(NODE_CHUNK, NODE_LAST - NODE_CHUNK)],
            out.at[pl.ds(NW * NODE_CHUNK, NODE_LAST - NODE_CHUNK)])


def _s3_call(bounds, adj0, adj1, ev, feats, vhat):
    f = pl.kernel(
        _s3_body,
        out_type=jax.ShapeDtypeStruct((N_NODE, N_DIM), jnp.float32),
        mesh=_mesh,
        compiler_params=pltpu.CompilerParams(needs_layout_passes=False),
        scratch_types=[
            pltpu.VMEM((LANES,), jnp.int32),
            pltpu.VMEM((TB + LANES,), jnp.int32),
            pltpu.VMEM((TB,), jnp.float32),
            pltpu.VMEM((TB,), jnp.int32),
            pltpu.VMEM((TB, N_DIM), jnp.float32),
            pltpu.VMEM((TB, N_DIM), jnp.float32),
            pltpu.VMEM((LANES,), jnp.float32),
            pltpu.VMEM((NODE_ACC, N_DIM), jnp.float32),
            pltpu.VMEM((NODE_ACC, LANES), jnp.float32),
            pltpu.SemaphoreType.DMA,
            pltpu.SemaphoreType.DMA,
            pltpu.SemaphoreType.DMA,
            pltpu.SemaphoreType.DMA,
            pltpu.SemaphoreType.DMA,
        ],
    )
    return f(bounds, adj0, adj1, ev, feats, vhat)


def _s4_body(bounds, hi0, hi1, pe, feats, out,
             bnd_v, seg_v, idx_v, pe_v, q_buf, f_rows, acc_v, s_v,
             sem, sem2, sem3):
    w = _worker_id()
    pltpu.sync_copy(bounds.at[w], bnd_v)
    bv = bnd_v[...]
    b0 = bv[0]
    b1 = bv[1]
    n0 = w * NODE_CHUNK
    pltpu.sync_copy(pe, pe_v.at[pl.ds(0, N_NODE)])
    _zero_rows(acc_v, NODE_ACC)
    _zero_svec(s_v, NODE_ACC)

    a8 = (b0 // 8) * 8
    nbat = (b1 - a8 + TB - 1) // TB

    def batch(k, bcarry):
        base = a8 + k * TB
        c1 = pltpu.async_copy(hi1.at[pl.ds(base, TB)], idx_v, sem2)
        c2 = pltpu.async_copy(
            hi0.at[pl.ds(base, TB)], seg_v.at[pl.ds(0, TB)], sem3)
        c1.wait()
        cg = pltpu.async_copy(feats.at[idx_v], f_rows, sem)

        def qg(g, qcarry):
            sv = idx_v[pl.ds(g * LANES, LANES)]
            q_buf[pl.ds(g * LANES, LANES)] = plsc.load_gather(pe_v, [sv])
            return qcarry
        lax.fori_loop(0, TB // LANES, qg, None)
        c2.wait()
        cg.wait()
        lo = jnp.maximum(b0 - base, 0)
        hi = jnp.minimum(b1 - base, TB)

        def edge(i, ecarry):
            d = seg_v[pl.ds(i, LANES)][0] - n0
            ibc = jnp.full((LANES,), i, jnp.int32)
            qb = plsc.load_gather(q_buf, [ibc])
            s_v[d, pl.ds(0, LANES)] = s_v[d, pl.ds(0, LANES)] + qb
            for c in range(DSL):
                sl = pl.ds(c * LANES, LANES)
                acc_v[d, sl] = acc_v[d, sl] + qb * f_rows[i, sl]
            return ecarry
        lax.fori_loop(lo, hi, edge, None)
        return bcarry
    lax.fori_loop(0, nbat, batch, None)

    def fin(r, carry):
        rsv = 1.0 / (s_v[r, pl.ds(0, LANES)] + 1e-16)
        for c in range(DSL):
            sl = pl.ds(c * LANES, LANES)
            acc_v[r, sl] = acc_v[r, sl] * rsv
        return carry
    lax.fori_loop(0, NODE_ACC, fin, None)

    pltpu.sync_copy(acc_v.at[pl.ds(0, NODE_CHUNK)], out.at[pl.ds(n0, NODE_CHUNK)])

    @pl.when(w == NW - 1)
    def _():
        pltpu.sync_copy(
            acc_v.at[pl.ds(NODE_CHUNK, NODE_LAST - NODE_CHUNK)],
            out.at[pl.ds(NW * NODE_CHUNK, NODE_LAST - NODE_CHUNK)])


def _s4_call(bounds, hi0, hi1, pe, feats):
    f = pl.kernel(
        _s4_body,
        out_type=jax.ShapeDtypeStruct((N_NODE, N_DIM), jnp.float32),
        mesh=_mesh,
        compiler_params=pltpu.CompilerParams(needs_layout_passes=False),
        scratch_types=[
            pltpu.VMEM((LANES,), jnp.int32),
            pltpu.VMEM((TB + LANES,), jnp.int32),
            pltpu.VMEM((TB,), jnp.int32),
            pltpu.VMEM((N_NODE + LANES,), jnp.float32),
            pltpu.VMEM((TB,), jnp.float32),
            pltpu.VMEM((TB, N_DIM), jnp.float32),
            pltpu.VMEM((NODE_ACC, N_DIM), jnp.float32),
            pltpu.VMEM((NODE_ACC, LANES), jnp.float32),
            pltpu.SemaphoreType.DMA,
            pltpu.SemaphoreType.DMA,
            pltpu.SemaphoreType.DMA,
        ],
    )
    return f(bounds, hi0, hi1, pe, feats)


def _tanh_body(x_ref, o_ref):
    o_ref[...] = jnp.tanh(x_ref[...])


def _tanh_call(x):
    return pl.pallas_call(
        _tanh_body,
        out_shape=jax.ShapeDtypeStruct(x.shape, jnp.float32),
    )(x)


def _t1_body(tri_ref, k01_ref, v_ref, e_ref):
    x = tri_ref[...]
    ss = jnp.sum(x * x, axis=1, keepdims=True)
    v = x / jnp.maximum(jnp.sqrt(ss), 1e-12)
    v_ref[...] = v
    e_ref[...] = jnp.exp(v @ k01_ref[...])


def _t1_call(tri, k01):
    n = tri.shape[0]
    br = 1024
    return pl.pallas_call(
        _t1_body,
        grid=(n // br,),
        in_specs=[pl.BlockSpec((br, N_DIM), lambda i: (i, 0)),
                  pl.BlockSpec((N_DIM, 2), lambda i: (0, 0))],
        out_specs=[pl.BlockSpec((br, N_DIM), lambda i: (i, 0)),
                   pl.BlockSpec((br, 2), lambda i: (i, 0))],
        out_shape=[jax.ShapeDtypeStruct((n, N_DIM), jnp.float32),
                   jax.ShapeDtypeStruct((n, 2), jnp.float32)],
    )(tri, k01)


def _t2pe_body(raw_ref, ha_ref, f_ref, pe_ref):
    f = jnp.tanh(raw_ref[...])
    f_ref[...] = f
    p = jnp.exp(f @ ha_ref[...])
    pe_ref[...] = p


def _t2pe_call(raw, ha):
    return pl.pallas_call(
        _t2pe_body,
        out_shape=[jax.ShapeDtypeStruct((N_NODE, N_DIM), jnp.float32),
                   jax.ShapeDtypeStruct((N_NODE, 1), jnp.float32)],
    )(raw, ha)


def kernel(features, rel_emb, adj, r_index, r_val, high_nei,
           attn_k0, attn_k1, high_att0, high_att1):
    r0, r1 = r_index[0], r_index[1]
    a0, a1 = adj[0], adj[1]
    h0, h1 = high_nei[0], high_nei[1]

    seg_grid = jnp.arange(N_SEG_CHUNKS + 1, dtype=jnp.int32) * SEG_CHUNK
    sb = jnp.searchsorted(r0, seg_grid, side="left").astype(jnp.int32)
    s1_bounds = jnp.zeros((NW * SEG_CHUNKS_PER_W, LANES), jnp.int32)
    s1_bounds = s1_bounds.at[:N_SEG_CHUNKS, 0].set(sb[:-1])
    s1_bounds = s1_bounds.at[:N_SEG_CHUNKS, 1].set(sb[1:])

    node_grid = jnp.concatenate([
        jnp.arange(NW, dtype=jnp.int32) * NODE_CHUNK,
        jnp.array([N_NODE], jnp.int32)])
    ab = jnp.searchsorted(a0, node_grid, side="left").astype(jnp.int32)
    s3_bounds = jnp.zeros((NW, LANES), jnp.int32)
    s3_bounds = s3_bounds.at[:, 0].set(ab[:-1]).at[:, 1].set(ab[1:])
    hb = jnp.searchsorted(h0, node_grid, side="left").astype(jnp.int32)
    s4_bounds = jnp.zeros((NW, LANES), jnp.int32)
    s4_bounds = s4_bounds.at[:, 0].set(hb[:-1]).at[:, 1].set(hb[1:])

    pad1 = lambda x: jnp.pad(x, (0, PAD))
    r0p, r1p, rvp = pad1(r0), pad1(r1), pad1(r_val)
    a0p, a1p = pad1(a0), pad1(a1)
    h0p, h1p = pad1(h0), pad1(h1)

    tri = _s1_call(s1_bounds, r1p, r0p, rvp, rel_emb)
    k01 = jnp.concatenate([attn_k0, attn_k1], axis=1)
    vhat, e01 = _t1_call(tri, k01)
    e0 = jnp.asarray(e01[:, 0])
    e1 = jnp.asarray(e01[:, 1])

    f0 = _tanh_call(features)
    raw1 = _s3_call(s3_bounds, a0p, a1p, e0, f0, vhat)
    f1 = _tanh_call(raw1)
    raw2 = _s3_call(s3_bounds, a0p, a1p, e1, f1, vhat)
    f2, pe0 = _t2pe_call(raw2, high_att0)
    raw3 = _s4_call(s4_bounds, h0p, h1p, jnp.squeeze(pe0, -1), f2)
    f3, pe1 = _t2pe_call(raw3, high_att1)
    raw4 = _s4_call(s4_bounds, h0p, h1p, jnp.squeeze(pe1, -1), f3)
    f4 = _tanh_call(raw4)
    return jnp.concatenate([f0, f1, f2, f3, f4], axis=-1)


# speedup vs baseline: 4.7330x; 1.1625x over previous
import jax
import jax.numpy as jnp
from jax import lax
from jax.experimental import pallas as pl
from jax.experimental.pallas import tpu as pltpu
from jax.experimental.pallas import tpu_sc as plsc

N_NODE = 10000
N_DIM = 128
N_EDGE = 320000
N_TRI = 320000
LANES = 16
DSL = N_DIM // LANES

NC, NS = 2, 16
NW = NC * NS

TB = 128
PAD = 512
SEG_CHUNK = 512
N_SEG_CHUNKS = N_TRI // SEG_CHUNK
SEG_CHUNKS_PER_W = -(-N_SEG_CHUNKS // NW)
NODE_CHUNK = 312
NODE_LAST = N_NODE - (NW - 1) * NODE_CHUNK
NODE_ACC = NODE_LAST

_mesh = plsc.VectorSubcoreMesh(
    core_axis_name="c", subcore_axis_name="s", num_cores=NC, num_subcores=NS)


def _worker_id():
    return lax.axis_index("s") * NC + lax.axis_index("c")


def _zero_rows(acc_v, nrows):
    z = jnp.zeros((LANES,), jnp.float32)

    def zrow(r, carry):
        for c in range(DSL):
            acc_v[r, pl.ds(c * LANES, LANES)] = z
        return carry
    lax.fori_loop(0, nrows, zrow, None)


def _zero_svec(s_v, n):
    z = jnp.zeros((LANES,), jnp.float32)

    def zs(r, carry):
        s_v[r, pl.ds(0, LANES)] = z
        return carry
    lax.fori_loop(0, n, zs, None)


def _s1_body(bounds, relids, segids, rval, rel_emb, tri_out,
             bnd_v, idx_v, seg_v, val_v, rows_v, acc_v, sem, sem2, sem3, sem4):
    w = _worker_id()

    def chunk(ci, carry):
        j = w + ci * NW

        @pl.when(j < N_SEG_CHUNKS)
        def _():
            pltpu.sync_copy(bounds.at[j], bnd_v)
            bv = bnd_v[...]
            t0 = bv[0]
            t1 = bv[1]
            seg_base = j * SEG_CHUNK
            _zero_rows(acc_v, SEG_CHUNK)
            a8 = (t0 // 8) * 8
            nbat = (t1 - a8 + TB - 1) // TB

            def batch(k, bcarry):
                base = a8 + k * TB
                c1 = pltpu.async_copy(relids.at[pl.ds(base, TB)], idx_v, sem2)
                c2 = pltpu.async_copy(
                    segids.at[pl.ds(base, TB)], seg_v.at[pl.ds(0, TB)], sem3)
                c3 = pltpu.async_copy(rval.at[pl.ds(base, TB)], val_v.at[pl.ds(0, TB)], sem4)
                c1.wait()
                cg = pltpu.async_copy(rel_emb.at[idx_v], rows_v, sem)
                c2.wait()
                c3.wait()
                cg.wait()
                lo = jnp.maximum(t0 - base, 0)
                hi = jnp.minimum(t1 - base, TB)

                def tri(i, tcarry):
                    s, vvb = tcarry
                    s_nx = seg_v[pl.ds(i + 1, LANES)][0] - seg_base
                    vvb_nx = plsc.load_gather(
                        val_v, [jnp.full((LANES,), i + 1, jnp.int32)])
                    for c in range(DSL):
                        sl = pl.ds(c * LANES, LANES)
                        acc_v[s, sl] = acc_v[s, sl] + vvb * rows_v[i, sl]
                    return (s_nx, vvb_nx)
                s0 = seg_v[pl.ds(lo, LANES)][0] - seg_base
                vvb0 = plsc.load_gather(
                    val_v, [jnp.full((LANES,), lo, jnp.int32)])
                lax.fori_loop(lo, hi, tri, (s0, vvb0))
                return bcarry
            lax.fori_loop(0, nbat, batch, None)
            pltpu.sync_copy(acc_v, tri_out.at[pl.ds(seg_base, SEG_CHUNK)])
        return carry
    lax.fori_loop(0, SEG_CHUNKS_PER_W, chunk, None)


def _s1_call(bounds, relids, segids, rval, rel_emb):
    f = pl.kernel(
        _s1_body,
        out_type=jax.ShapeDtypeStruct((N_TRI + PAD, N_DIM), jnp.float32),
        mesh=_mesh,
        compiler_params=pltpu.CompilerParams(needs_layout_passes=False),
        scratch_types=[
            pltpu.VMEM((LANES,), jnp.int32),
            pltpu.VMEM((TB,), jnp.int32),
            pltpu.VMEM((TB + LANES,), jnp.int32),
            pltpu.VMEM((TB + LANES,), jnp.float32),
            pltpu.VMEM((TB, N_DIM), jnp.float32),
            pltpu.VMEM((SEG_CHUNK, N_DIM), jnp.float32),
            pltpu.SemaphoreType.DMA,
            pltpu.SemaphoreType.DMA,
            pltpu.SemaphoreType.DMA,
            pltpu.SemaphoreType.DMA,
        ],
    )
    return f(bounds, relids, segids, rval, rel_emb)


def _s3_body(bounds, adj0, adj1, ev, feats, vhat, out,
             bnd_v, seg_v, e_v, idx_v, f_rows, v_rows, dot_scr, acc_v, s_v,
             sem, sem2, sem3, sem4, sem5):
    w = _worker_id()
    pltpu.sync_copy(bounds.at[w], bnd_v)
    bv = bnd_v[...]
    b0 = bv[0]
    b1 = bv[1]
    n0 = w * NODE_CHUNK
    _zero_rows(acc_v, NODE_ACC)
    _zero_svec(s_v, NODE_ACC)
    io = lax.iota(jnp.int32, LANES)

    a8 = (b0 // 8) * 8
    nbat = (b1 - a8 + TB - 1) // TB

    def batch(k, bcarry):
        base = a8 + k * TB
        c1 = pltpu.async_copy(adj1.at[pl.ds(base, TB)], idx_v, sem2)
        c2 = pltpu.async_copy(
            adj0.at[pl.ds(base, TB)], seg_v.at[pl.ds(0, TB)], sem3)
        c3 = pltpu.async_copy(ev.at[pl.ds(base, TB)], e_v.at[pl.ds(0, TB)], sem4)
        c4 = pltpu.async_copy(vhat.at[pl.ds(base, TB)], v_rows, sem5)
        c1.wait()
        cg = pltpu.async_copy(feats.at[idx_v], f_rows, sem)
        c2.wait()
        c3.wait()
        c4.wait()
        cg.wait()
        lo = jnp.maximum(b0 - base, 0)
        hi = jnp.minimum(b1 - base, TB)

        def edge(i, ecarry):
            d, eeb = ecarry
            d_nx = seg_v[pl.ds(i + 1, LANES)][0] - n0
            eeb_nx = plsc.load_gather(
                e_v, [jnp.full((LANES,), i + 1, jnp.int32)])
            s_v[d, pl.ds(0, LANES)] = s_v[d, pl.ds(0, LANES)] + eeb
            p0 = f_rows[i, pl.ds(0, LANES)] * v_rows[i, pl.ds(0, LANES)]
            p1 = f_rows[i, pl.ds(LANES, LANES)] * v_rows[i, pl.ds(LANES, LANES)]
            for c in range(2, DSL, 2):
                sl = pl.ds(c * LANES, LANES)
                sl2 = pl.ds((c + 1) * LANES, LANES)
                p0 = p0 + f_rows[i, sl] * v_rows[i, sl]
                p1 = p1 + f_rows[i, sl2] * v_rows[i, sl2]
            x = p0 + p1
            for kk in (8, 4, 2, 1):
                dot_scr[...] = x
                x = x + plsc.load_gather(dot_scr, [io ^ kk])
            coefb = x * (-2.0 * eeb)
            for c in range(DSL):
                sl = pl.ds(c * LANES, LANES)
                acc_v[d, sl] = acc_v[d, sl] + eeb * f_rows[i, sl] + coefb * v_rows[i, sl]
            return (d_nx, eeb_nx)
        d0 = seg_v[pl.ds(lo, LANES)][0] - n0
        eeb0 = plsc.load_gather(e_v, [jnp.full((LANES,), lo, jnp.int32)])
        lax.fori_loop(lo, hi, edge, (d0, eeb0))
        return bcarry
    lax.fori_loop(0, nbat, batch, None)

    def fin(r, carry):
        rsv = 1.0 / (s_v[r, pl.ds(0, LANES)] + 1e-16)
        for c in range(DSL):
            sl = pl.ds(c * LANES, LANES)
            acc_v[r, sl] = acc_v[r, sl] * rsv
        return carry
    lax.fori_loop(0, NODE_ACC, fin, None)

    pltpu.sync_copy(acc_v.at[pl.ds(0, NODE_CHUNK)], out.at[pl.ds(n0, NODE_CHUNK)])

    @pl.when(w == NW - 1)
    def _():
        pltpu.sync_copy(
            acc_v.at[pl.ds(NODE_CHUNK, NODE_LAST - NODE_CHUNK)],
            out.at[pl.ds(NW * NODE_CHUNK, NODE_LAST - NODE_CHUNK)])


def _s3_call(bounds, adj0, adj1, ev, feats, vhat):
    f = pl.kernel(
        _s3_body,
        out_type=jax.ShapeDtypeStruct((N_NODE, N_DIM), jnp.float32),
        mesh=_mesh,
        compiler_params=pltpu.CompilerParams(needs_layout_passes=False),
        scratch_types=[
            pltpu.VMEM((LANES,), jnp.int32),
            pltpu.VMEM((TB + LANES,), jnp.int32),
            pltpu.VMEM((TB + LANES,), jnp.float32),
            pltpu.VMEM((TB,), jnp.int32),
            pltpu.VMEM((TB, N_DIM), jnp.float32),
            pltpu.VMEM((TB, N_DIM), jnp.float32),
            pltpu.VMEM((LANES,), jnp.float32),
            pltpu.VMEM((NODE_ACC, N_DIM), jnp.float32),
            pltpu.VMEM((NODE_ACC, LANES), jnp.float32),
            pltpu.SemaphoreType.DMA,
            pltpu.SemaphoreType.DMA,
            pltpu.SemaphoreType.DMA,
            pltpu.SemaphoreType.DMA,
            pltpu.SemaphoreType.DMA,
        ],
    )
    return f(bounds, adj0, adj1, ev, feats, vhat)


def _s4_body(bounds, hi0, hi1, pe, feats, out,
             bnd_v, seg_v, idx_v, pe_v, q_buf, f_rows, acc_v, s_v,
             sem, sem2, sem3):
    w = _worker_id()
    pltpu.sync_copy(bounds.at[w], bnd_v)
    bv = bnd_v[...]
    b0 = bv[0]
    b1 = bv[1]
    n0 = w * NODE_CHUNK
    pltpu.sync_copy(pe, pe_v.at[pl.ds(0, N_NODE)])
    _zero_rows(acc_v, NODE_ACC)
    _zero_svec(s_v, NODE_ACC)

    a8 = (b0 // 8) * 8
    nbat = (b1 - a8 + TB - 1) // TB

    def batch(k, bcarry):
        base = a8 + k * TB
        c1 = pltpu.async_copy(hi1.at[pl.ds(base, TB)], idx_v, sem2)
        c2 = pltpu.async_copy(
            hi0.at[pl.ds(base, TB)], seg_v.at[pl.ds(0, TB)], sem3)
        c1.wait()
        cg = pltpu.async_copy(feats.at[idx_v], f_rows, sem)

        def qg(g, qcarry):
            sv = idx_v[pl.ds(g * LANES, LANES)]
            q_buf[pl.ds(g * LANES, LANES)] = plsc.load_gather(pe_v, [sv])
            return qcarry
        lax.fori_loop(0, TB // LANES, qg, None)
        c2.wait()
        cg.wait()
        lo = jnp.maximum(b0 - base, 0)
        hi = jnp.minimum(b1 - base, TB)

        def edge(i, ecarry):
            d, qb = ecarry
            d_nx = seg_v[pl.ds(i + 1, LANES)][0] - n0
            qb_nx = plsc.load_gather(
                q_buf, [jnp.full((LANES,), i + 1, jnp.int32)])
            s_v[d, pl.ds(0, LANES)] = s_v[d, pl.ds(0, LANES)] + qb
            for c in range(DSL):
                sl = pl.ds(c * LANES, LANES)
                acc_v[d, sl] = acc_v[d, sl] + qb * f_rows[i, sl]
            return (d_nx, qb_nx)
        d0 = seg_v[pl.ds(lo, LANES)][0] - n0
        qb0 = plsc.load_gather(q_buf, [jnp.full((LANES,), lo, jnp.int32)])
        lax.fori_loop(lo, hi, edge, (d0, qb0))
        return bcarry
    lax.fori_loop(0, nbat, batch, None)

    def fin(r, carry):
        rsv = 1.0 / (s_v[r, pl.ds(0, LANES)] + 1e-16)
        for c in range(DSL):
            sl = pl.ds(c * LANES, LANES)
            acc_v[r, sl] = acc_v[r, sl] * rsv
        return carry
    lax.fori_loop(0, NODE_ACC, fin, None)

    pltpu.sync_copy(acc_v.at[pl.ds(0, NODE_CHUNK)], out.at[pl.ds(n0, NODE_CHUNK)])

    @pl.when(w == NW - 1)
    def _():
        pltpu.sync_copy(
            acc_v.at[pl.ds(NODE_CHUNK, NODE_LAST - NODE_CHUNK)],
            out.at[pl.ds(NW * NODE_CHUNK, NODE_LAST - NODE_CHUNK)])


def _s4_call(bounds, hi0, hi1, pe, feats):
    f = pl.kernel(
        _s4_body,
        out_type=jax.ShapeDtypeStruct((N_NODE, N_DIM), jnp.float32),
        mesh=_mesh,
        compiler_params=pltpu.CompilerParams(needs_layout_passes=False),
        scratch_types=[
            pltpu.VMEM((LANES,), jnp.int32),
            pltpu.VMEM((TB + LANES,), jnp.int32),
            pltpu.VMEM((TB,), jnp.int32),
            pltpu.VMEM((N_NODE + LANES,), jnp.float32),
            pltpu.VMEM((TB + LANES,), jnp.float32),
            pltpu.VMEM((TB, N_DIM), jnp.float32),
            pltpu.VMEM((NODE_ACC, N_DIM), jnp.float32),
            pltpu.VMEM((NODE_ACC, LANES), jnp.float32),
            pltpu.SemaphoreType.DMA,
            pltpu.SemaphoreType.DMA,
            pltpu.SemaphoreType.DMA,
        ],
    )
    return f(bounds, hi0, hi1, pe, feats)


def _tanh_body(x_ref, o_ref):
    o_ref[...] = jnp.tanh(x_ref[...])


def _tanh_call(x):
    return pl.pallas_call(
        _tanh_body,
        out_shape=jax.ShapeDtypeStruct(x.shape, jnp.float32),
    )(x)


def _t1_body(tri_ref, k01_ref, v_ref, e_ref):
    x = tri_ref[...]
    ss = jnp.sum(x * x, axis=1, keepdims=True)
    v = x / jnp.maximum(jnp.sqrt(ss), 1e-12)
    v_ref[...] = v
    e_ref[...] = jnp.exp(v @ k01_ref[...])


def _t1_call(tri, k01):
    n = tri.shape[0]
    br = 1024
    return pl.pallas_call(
        _t1_body,
        grid=(n // br,),
        in_specs=[pl.BlockSpec((br, N_DIM), lambda i: (i, 0)),
                  pl.BlockSpec((N_DIM, 2), lambda i: (0, 0))],
        out_specs=[pl.BlockSpec((br, N_DIM), lambda i: (i, 0)),
                   pl.BlockSpec((br, 2), lambda i: (i, 0))],
        out_shape=[jax.ShapeDtypeStruct((n, N_DIM), jnp.float32),
                   jax.ShapeDtypeStruct((n, 2), jnp.float32)],
    )(tri, k01)


def _t2pe_body(raw_ref, ha_ref, f_ref, pe_ref):
    f = jnp.tanh(raw_ref[...])
    f_ref[...] = f
    p = jnp.exp(f @ ha_ref[...])
    pe_ref[...] = p


def _t2pe_call(raw, ha):
    return pl.pallas_call(
        _t2pe_body,
        out_shape=[jax.ShapeDtypeStruct((N_NODE, N_DIM), jnp.float32),
                   jax.ShapeDtypeStruct((N_NODE, 1), jnp.float32)],
    )(raw, ha)


def kernel(features, rel_emb, adj, r_index, r_val, high_nei,
           attn_k0, attn_k1, high_att0, high_att1):
    r0, r1 = r_index[0], r_index[1]
    a0, a1 = adj[0], adj[1]
    h0, h1 = high_nei[0], high_nei[1]

    seg_grid = jnp.arange(N_SEG_CHUNKS + 1, dtype=jnp.int32) * SEG_CHUNK
    sb = jnp.searchsorted(r0, seg_grid, side="left").astype(jnp.int32)
    s1_bounds = jnp.zeros((NW * SEG_CHUNKS_PER_W, LANES), jnp.int32)
    s1_bounds = s1_bounds.at[:N_SEG_CHUNKS, 0].set(sb[:-1])
    s1_bounds = s1_bounds.at[:N_SEG_CHUNKS, 1].set(sb[1:])

    node_grid = jnp.concatenate([
        jnp.arange(NW, dtype=jnp.int32) * NODE_CHUNK,
        jnp.array([N_NODE], jnp.int32)])
    ab = jnp.searchsorted(a0, node_grid, side="left").astype(jnp.int32)
    s3_bounds = jnp.zeros((NW, LANES), jnp.int32)
    s3_bounds = s3_bounds.at[:, 0].set(ab[:-1]).at[:, 1].set(ab[1:])
    hb = jnp.searchsorted(h0, node_grid, side="left").astype(jnp.int32)
    s4_bounds = jnp.zeros((NW, LANES), jnp.int32)
    s4_bounds = s4_bounds.at[:, 0].set(hb[:-1]).at[:, 1].set(hb[1:])

    pad1 = lambda x: jnp.pad(x, (0, PAD))
    r0p, r1p, rvp = pad1(r0), pad1(r1), pad1(r_val)
    a0p, a1p = pad1(a0), pad1(a1)
    h0p, h1p = pad1(h0), pad1(h1)

    tri = _s1_call(s1_bounds, r1p, r0p, rvp, rel_emb)
    k01 = jnp.concatenate([attn_k0, attn_k1], axis=1)
    vhat, e01 = _t1_call(tri, k01)
    e0 = jnp.asarray(e01[:, 0])
    e1 = jnp.asarray(e01[:, 1])

    f0 = _tanh_call(features)
    raw1 = _s3_call(s3_bounds, a0p, a1p, e0, f0, vhat)
    f1 = _tanh_call(raw1)
    raw2 = _s3_call(s3_bounds, a0p, a1p, e1, f1, vhat)
    f2, pe0 = _t2pe_call(raw2, high_att0)
    raw3 = _s4_call(s4_bounds, h0p, h1p, jnp.squeeze(pe0, -1), f2)
    f3, pe1 = _t2pe_call(raw3, high_att1)
    raw4 = _s4_call(s4_bounds, h0p, h1p, jnp.squeeze(pe1, -1), f3)
    f4 = _tanh_call(raw4)
    return jnp.concatenate([f0, f1, f2, f3, f4], axis=-1)


# speedup vs baseline: 6.3871x; 1.3495x over previous
import jax
import jax.numpy as jnp
from jax import lax
from jax.experimental import pallas as pl
from jax.experimental.pallas import tpu as pltpu
from jax.experimental.pallas import tpu_sc as plsc

N_NODE = 10000
N_DIM = 128
N_EDGE = 320000
N_TRI = 320000
LANES = 16
DSL = N_DIM // LANES

NC, NS = 2, 16
NW = NC * NS

TB = 128
PAD = 512
SEG_CHUNK = 512
N_SEG_CHUNKS = N_TRI // SEG_CHUNK
SEG_CHUNKS_PER_W = -(-N_SEG_CHUNKS // NW)
NODE_CHUNK = 312
NODE_LAST = N_NODE - (NW - 1) * NODE_CHUNK
NODE_ACC = NODE_LAST

_mesh = plsc.VectorSubcoreMesh(
    core_axis_name="c", subcore_axis_name="s", num_cores=NC, num_subcores=NS)


def _worker_id():
    return lax.axis_index("s") * NC + lax.axis_index("c")


def _zero_rows(acc_v, nrows):
    z = jnp.zeros((LANES,), jnp.float32)

    def zrow(r, carry):
        for c in range(DSL):
            acc_v[r, pl.ds(c * LANES, LANES)] = z
        return carry
    lax.fori_loop(0, nrows, zrow, None)


def _s1_body(bounds, relids, segids, rval, rel_emb, tri_out,
             bnd_v, idx_v, seg_v, val_v, rows_v, acc_v, sem, sem2, sem3, sem4):
    w = _worker_id()

    def chunk(ci, carry):
        j = w + ci * NW

        @pl.when(j < N_SEG_CHUNKS)
        def _():
            pltpu.sync_copy(bounds.at[j], bnd_v)
            bv = bnd_v[...]
            t0 = bv[0]
            t1 = bv[1]
            seg_base = j * SEG_CHUNK
            _zero_rows(acc_v, SEG_CHUNK)
            a8 = (t0 // 8) * 8
            nbat = (t1 - a8 + TB - 1) // TB

            def batch(k, bcarry):
                base = a8 + k * TB
                c1 = pltpu.async_copy(relids.at[pl.ds(base, TB)], idx_v, sem2)
                c2 = pltpu.async_copy(
                    segids.at[pl.ds(base, TB)], seg_v.at[pl.ds(0, TB)], sem3)
                c3 = pltpu.async_copy(rval.at[pl.ds(base, TB)], val_v.at[pl.ds(0, TB)], sem4)
                c1.wait()
                cg = pltpu.async_copy(rel_emb.at[idx_v], rows_v, sem)
                c2.wait()
                c3.wait()
                cg.wait()
                lo = jnp.maximum(t0 - base, 0)
                hi = jnp.minimum(t1 - base, TB)

                def tri(i, tcarry):
                    s, vvb = tcarry
                    s_nx = seg_v[pl.ds(i + 1, LANES)][0] - seg_base
                    vvb_nx = plsc.load_gather(
                        val_v, [jnp.full((LANES,), i + 1, jnp.int32)])
                    for c in range(DSL):
                        sl = pl.ds(c * LANES, LANES)
                        acc_v[s, sl] = acc_v[s, sl] + vvb * rows_v[i, sl]
                    return (s_nx, vvb_nx)
                s0 = seg_v[pl.ds(lo, LANES)][0] - seg_base
                vvb0 = plsc.load_gather(
                    val_v, [jnp.full((LANES,), lo, jnp.int32)])
                lax.fori_loop(lo, hi, tri, (s0, vvb0))
                return bcarry
            lax.fori_loop(0, nbat, batch, None)
            pltpu.sync_copy(acc_v, tri_out.at[pl.ds(seg_base, SEG_CHUNK)])
        return carry
    lax.fori_loop(0, SEG_CHUNKS_PER_W, chunk, None)


def _s1_call(bounds, relids, segids, rval, rel_emb):
    f = pl.kernel(
        _s1_body,
        out_type=jax.ShapeDtypeStruct((N_TRI + PAD, N_DIM), jnp.float32),
        mesh=_mesh,
        compiler_params=pltpu.CompilerParams(needs_layout_passes=False),
        scratch_types=[
            pltpu.VMEM((LANES,), jnp.int32),
            pltpu.VMEM((TB,), jnp.int32),
            pltpu.VMEM((TB + LANES,), jnp.int32),
            pltpu.VMEM((TB + LANES,), jnp.float32),
            pltpu.VMEM((TB, N_DIM), jnp.float32),
            pltpu.VMEM((SEG_CHUNK, N_DIM), jnp.float32),
            pltpu.SemaphoreType.DMA,
            pltpu.SemaphoreType.DMA,
            pltpu.SemaphoreType.DMA,
            pltpu.SemaphoreType.DMA,
        ],
    )
    return f(bounds, relids, segids, rval, rel_emb)


def _s3_body(bounds, adj0, adj1, ev, feats, vhat, out,
             bnd_v, seg_v, e_v, idx_v, f_rows, v_rows, dot_buf, dots_v,
             acc_v, sem, sem2, sem3, sem4, sem5):
    w = _worker_id()
    pltpu.sync_copy(bounds.at[w], bnd_v)
    bv = bnd_v[...]
    b0 = bv[0]
    b1 = bv[1]
    n0 = w * NODE_CHUNK
    _zero_rows(acc_v, NODE_ACC)
    io = lax.iota(jnp.int32, LANES)
    zv = jnp.zeros((LANES,), jnp.float32)

    a8 = (b0 // 8) * 8
    nbat = (b1 - a8 + TB - 1) // TB

    def batch(k, bcarry):
        d_cur = bcarry[0]
        sreg = bcarry[1]
        accs = bcarry[2:]
        base = a8 + k * TB
        c1 = pltpu.async_copy(adj1.at[pl.ds(base, TB)], idx_v, sem2)
        c2 = pltpu.async_copy(
            adj0.at[pl.ds(base, TB)], seg_v.at[pl.ds(0, TB)], sem3)
        c3 = pltpu.async_copy(ev.at[pl.ds(base, TB)], e_v.at[pl.ds(0, TB)], sem4)
        c4 = pltpu.async_copy(vhat.at[pl.ds(base, TB)], v_rows, sem5)
        c1.wait()
        cg = pltpu.async_copy(feats.at[idx_v], f_rows, sem)
        c2.wait()
        c3.wait()
        c4.wait()
        cg.wait()
        lo = jnp.maximum(b0 - base, 0)
        hi = jnp.minimum(b1 - base, TB)

        def dgrp(g, dcarry):
            for u in range(LANES):
                i = g * LANES + u
                p0 = f_rows[i, pl.ds(0, LANES)] * v_rows[i, pl.ds(0, LANES)]
                p1 = f_rows[i, pl.ds(LANES, LANES)] * v_rows[i, pl.ds(LANES, LANES)]
                for c in range(2, DSL, 2):
                    sl = pl.ds(c * LANES, LANES)
                    sl2 = pl.ds((c + 1) * LANES, LANES)
                    p0 = p0 + f_rows[i, sl] * v_rows[i, sl]
                    p1 = p1 + f_rows[i, sl2] * v_rows[i, sl2]
                dot_buf[u, pl.ds(0, LANES)] = p0 + p1
            cols = [plsc.load_gather(dot_buf, [io, jnp.full((LANES,), c, jnp.int32)])
                    for c in range(LANES)]
            while len(cols) > 1:
                cols = [cols[t] + cols[t + 1] for t in range(0, len(cols), 2)]
            dots_v[pl.ds(g * LANES, LANES)] = cols[0]
            return dcarry
        lax.fori_loop(0, TB // LANES, dgrp, None)

        def edge(i, ecarry):
            dc = ecarry[0]
            sr = ecarry[1]
            ac = ecarry[2:]
            d_i = seg_v[pl.ds(i, LANES)][0] - n0
            ibc = jnp.full((LANES,), i, jnp.int32)
            eeb = plsc.load_gather(e_v, [ibc])
            dotb = plsc.load_gather(dots_v, [ibc])
            coefb = dotb * (-2.0 * eeb)
            flush = d_i != dc

            @pl.when(flush & (dc >= 0))
            def _():
                rsv = 1.0 / (sr + 1e-16)
                for c in range(DSL):
                    sl = pl.ds(c * LANES, LANES)
                    acc_v[dc, sl] = ac[c] * rsv

            new_ac = []
            for c in range(DSL):
                sl = pl.ds(c * LANES, LANES)
                contrib = eeb * f_rows[i, sl] + coefb * v_rows[i, sl]
                new_ac.append(jnp.where(flush, contrib, ac[c] + contrib))
            sr_new = jnp.where(flush, eeb, sr + eeb)
            return (d_i, sr_new) + tuple(new_ac)
        ecarry = lax.fori_loop(lo, hi, edge, (d_cur, sreg) + accs)
        return ecarry
    fcarry = lax.fori_loop(
        0, nbat, batch, (-1, zv) + (zv,) * DSL)
    d_cur = fcarry[0]
    sreg = fcarry[1]
    accs = fcarry[2:]

    @pl.when(d_cur >= 0)
    def _():
        rsv = 1.0 / (sreg + 1e-16)
        for c in range(DSL):
            sl = pl.ds(c * LANES, LANES)
            acc_v[d_cur, sl] = accs[c] * rsv

    pltpu.sync_copy(acc_v.at[pl.ds(0, NODE_CHUNK)], out.at[pl.ds(n0, NODE_CHUNK)])

    @pl.when(w == NW - 1)
    def _():
        pltpu.sync_copy(
            acc_v.at[pl.ds(NODE_CHUNK, NODE_LAST - NODE_CHUNK)],
            out.at[pl.ds(NW * NODE_CHUNK, NODE_LAST - NODE_CHUNK)])


def _s3_call(bounds, adj0, adj1, ev, feats, vhat):
    f = pl.kernel(
        _s3_body,
        out_type=jax.ShapeDtypeStruct((N_NODE, N_DIM), jnp.float32),
        mesh=_mesh,
        compiler_params=pltpu.CompilerParams(needs_layout_passes=False),
        scratch_types=[
            pltpu.VMEM((LANES,), jnp.int32),
            pltpu.VMEM((TB + LANES,), jnp.int32),
            pltpu.VMEM((TB + LANES,), jnp.float32),
            pltpu.VMEM((TB,), jnp.int32),
            pltpu.VMEM((TB, N_DIM), jnp.float32),
            pltpu.VMEM((TB, N_DIM), jnp.float32),
            pltpu.VMEM((LANES, LANES), jnp.float32),
            pltpu.VMEM((TB + LANES,), jnp.float32),
            pltpu.VMEM((NODE_ACC, N_DIM), jnp.float32),
            pltpu.SemaphoreType.DMA,
            pltpu.SemaphoreType.DMA,
            pltpu.SemaphoreType.DMA,
            pltpu.SemaphoreType.DMA,
            pltpu.SemaphoreType.DMA,
        ],
    )
    return f(bounds, adj0, adj1, ev, feats, vhat)


def _s4_body(bounds, hi0, hi1, pe, feats, out,
             bnd_v, seg_v, idx_v, pe_v, q_buf, f_rows, acc_v,
             sem, sem2, sem3):
    w = _worker_id()
    pltpu.sync_copy(bounds.at[w], bnd_v)
    bv = bnd_v[...]
    b0 = bv[0]
    b1 = bv[1]
    n0 = w * NODE_CHUNK
    pltpu.sync_copy(pe, pe_v.at[pl.ds(0, N_NODE)])
    _zero_rows(acc_v, NODE_ACC)

    a8 = (b0 // 8) * 8
    nbat = (b1 - a8 + TB - 1) // TB

    zv = jnp.zeros((LANES,), jnp.float32)

    def batch(k, bcarry):
        d_cur = bcarry[0]
        sreg = bcarry[1]
        accs = bcarry[2:]
        base = a8 + k * TB
        c1 = pltpu.async_copy(hi1.at[pl.ds(base, TB)], idx_v, sem2)
        c2 = pltpu.async_copy(
            hi0.at[pl.ds(base, TB)], seg_v.at[pl.ds(0, TB)], sem3)
        c1.wait()
        cg = pltpu.async_copy(feats.at[idx_v], f_rows, sem)

        def qg(g, qcarry):
            sv = idx_v[pl.ds(g * LANES, LANES)]
            q_buf[pl.ds(g * LANES, LANES)] = plsc.load_gather(pe_v, [sv])
            return qcarry
        lax.fori_loop(0, TB // LANES, qg, None)
        c2.wait()
        cg.wait()
        lo = jnp.maximum(b0 - base, 0)
        hi = jnp.minimum(b1 - base, TB)

        def edge(i, ecarry):
            dc = ecarry[0]
            sr = ecarry[1]
            ac = ecarry[2:]
            d_i = seg_v[pl.ds(i, LANES)][0] - n0
            qb = plsc.load_gather(q_buf, [jnp.full((LANES,), i, jnp.int32)])
            flush = d_i != dc

            @pl.when(flush & (dc >= 0))
            def _():
                rsv = 1.0 / (sr + 1e-16)
                for c in range(DSL):
                    sl = pl.ds(c * LANES, LANES)
                    acc_v[dc, sl] = ac[c] * rsv

            new_ac = []
            for c in range(DSL):
                sl = pl.ds(c * LANES, LANES)
                contrib = qb * f_rows[i, sl]
                new_ac.append(jnp.where(flush, contrib, ac[c] + contrib))
            sr_new = jnp.where(flush, qb, sr + qb)
            return (d_i, sr_new) + tuple(new_ac)
        ecarry = lax.fori_loop(lo, hi, edge, (d_cur, sreg) + accs)
        return ecarry
    fcarry = lax.fori_loop(0, nbat, batch, (-1, zv) + (zv,) * DSL)
    d_cur = fcarry[0]
    sreg = fcarry[1]
    accs = fcarry[2:]

    @pl.when(d_cur >= 0)
    def _():
        rsv = 1.0 / (sreg + 1e-16)
        for c in range(DSL):
            sl = pl.ds(c * LANES, LANES)
            acc_v[d_cur, sl] = accs[c] * rsv

    pltpu.sync_copy(acc_v.at[pl.ds(0, NODE_CHUNK)], out.at[pl.ds(n0, NODE_CHUNK)])

    @pl.when(w == NW - 1)
    def _():
        pltpu.sync_copy(
            acc_v.at[pl.ds(NODE_CHUNK, NODE_LAST - NODE_CHUNK)],
            out.at[pl.ds(NW * NODE_CHUNK, NODE_LAST - NODE_CHUNK)])


def _s4_call(bounds, hi0, hi1, pe, feats):
    f = pl.kernel(
        _s4_body,
        out_type=jax.ShapeDtypeStruct((N_NODE, N_DIM), jnp.float32),
        mesh=_mesh,
        compiler_params=pltpu.CompilerParams(needs_layout_passes=False),
        scratch_types=[
            pltpu.VMEM((LANES,), jnp.int32),
            pltpu.VMEM((TB + LANES,), jnp.int32),
            pltpu.VMEM((TB,), jnp.int32),
            pltpu.VMEM((N_NODE + LANES,), jnp.float32),
            pltpu.VMEM((TB + LANES,), jnp.float32),
            pltpu.VMEM((TB, N_DIM), jnp.float32),
            pltpu.VMEM((NODE_ACC, N_DIM), jnp.float32),
            pltpu.SemaphoreType.DMA,
            pltpu.SemaphoreType.DMA,
            pltpu.SemaphoreType.DMA,
        ],
    )
    return f(bounds, hi0, hi1, pe, feats)


def _tanh_body(x_ref, o_ref):
    o_ref[...] = jnp.tanh(x_ref[...])


def _tanh_call(x):
    return pl.pallas_call(
        _tanh_body,
        out_shape=jax.ShapeDtypeStruct(x.shape, jnp.float32),
    )(x)


def _t1_body(tri_ref, k01_ref, v_ref, e_ref):
    x = tri_ref[...]
    ss = jnp.sum(x * x, axis=1, keepdims=True)
    v = x / jnp.maximum(jnp.sqrt(ss), 1e-12)
    v_ref[...] = v
    e_ref[...] = jnp.exp(v @ k01_ref[...])


def _t1_call(tri, k01):
    n = tri.shape[0]
    br = 1024
    return pl.pallas_call(
        _t1_body,
        grid=(n // br,),
        in_specs=[pl.BlockSpec((br, N_DIM), lambda i: (i, 0)),
                  pl.BlockSpec((N_DIM, 2), lambda i: (0, 0))],
        out_specs=[pl.BlockSpec((br, N_DIM), lambda i: (i, 0)),
                   pl.BlockSpec((br, 2), lambda i: (i, 0))],
        out_shape=[jax.ShapeDtypeStruct((n, N_DIM), jnp.float32),
                   jax.ShapeDtypeStruct((n, 2), jnp.float32)],
    )(tri, k01)


def _t2pe_body(raw_ref, ha_ref, f_ref, pe_ref):
    f = jnp.tanh(raw_ref[...])
    f_ref[...] = f
    p = jnp.exp(f @ ha_ref[...])
    pe_ref[...] = p


def _t2pe_call(raw, ha):
    return pl.pallas_call(
        _t2pe_body,
        out_shape=[jax.ShapeDtypeStruct((N_NODE, N_DIM), jnp.float32),
                   jax.ShapeDtypeStruct((N_NODE, 1), jnp.float32)],
    )(raw, ha)


def kernel(features, rel_emb, adj, r_index, r_val, high_nei,
           attn_k0, attn_k1, high_att0, high_att1):
    r0, r1 = r_index[0], r_index[1]
    a0, a1 = adj[0], adj[1]
    h0, h1 = high_nei[0], high_nei[1]

    seg_grid = jnp.arange(N_SEG_CHUNKS + 1, dtype=jnp.int32) * SEG_CHUNK
    sb = jnp.searchsorted(r0, seg_grid, side="left").astype(jnp.int32)
    s1_bounds = jnp.zeros((NW * SEG_CHUNKS_PER_W, LANES), jnp.int32)
    s1_bounds = s1_bounds.at[:N_SEG_CHUNKS, 0].set(sb[:-1])
    s1_bounds = s1_bounds.at[:N_SEG_CHUNKS, 1].set(sb[1:])

    node_grid = jnp.concatenate([
        jnp.arange(NW, dtype=jnp.int32) * NODE_CHUNK,
        jnp.array([N_NODE], jnp.int32)])
    ab = jnp.searchsorted(a0, node_grid, side="left").astype(jnp.int32)
    s3_bounds = jnp.zeros((NW, LANES), jnp.int32)
    s3_bounds = s3_bounds.at[:, 0].set(ab[:-1]).at[:, 1].set(ab[1:])
    hb = jnp.searchsorted(h0, node_grid, side="left").astype(jnp.int32)
    s4_bounds = jnp.zeros((NW, LANES), jnp.int32)
    s4_bounds = s4_bounds.at[:, 0].set(hb[:-1]).at[:, 1].set(hb[1:])

    pad1 = lambda x: jnp.pad(x, (0, PAD))
    r0p, r1p, rvp = pad1(r0), pad1(r1), pad1(r_val)
    a0p, a1p = pad1(a0), pad1(a1)
    h0p, h1p = pad1(h0), pad1(h1)

    tri = _s1_call(s1_bounds, r1p, r0p, rvp, rel_emb)
    k01 = jnp.concatenate([attn_k0, attn_k1], axis=1)
    vhat, e01 = _t1_call(tri, k01)
    e0 = jnp.asarray(e01[:, 0])
    e1 = jnp.asarray(e01[:, 1])

    f0 = _tanh_call(features)
    raw1 = _s3_call(s3_bounds, a0p, a1p, e0, f0, vhat)
    f1 = _tanh_call(raw1)
    raw2 = _s3_call(s3_bounds, a0p, a1p, e1, f1, vhat)
    f2, pe0 = _t2pe_call(raw2, high_att0)
    raw3 = _s4_call(s4_bounds, h0p, h1p, jnp.squeeze(pe0, -1), f2)
    f3, pe1 = _t2pe_call(raw3, high_att1)
    raw4 = _s4_call(s4_bounds, h0p, h1p, jnp.squeeze(pe1, -1), f3)
    f4 = _tanh_call(raw4)
    return jnp.concatenate([f0, f1, f2, f3, f4], axis=-1)


# speedup vs baseline: 6.4198x; 1.0051x over previous
import jax
import jax.numpy as jnp
from jax import lax
from jax.experimental import pallas as pl
from jax.experimental.pallas import tpu as pltpu
from jax.experimental.pallas import tpu_sc as plsc

N_NODE = 10000
N_DIM = 128
N_EDGE = 320000
N_TRI = 320000
LANES = 16
DSL = N_DIM // LANES

NC, NS = 2, 16
NW = NC * NS

TB = 128
PAD = 512
SEG_CHUNK = 512
N_SEG_CHUNKS = N_TRI // SEG_CHUNK
SEG_CHUNKS_PER_W = -(-N_SEG_CHUNKS // NW)
NODE_CHUNK = 312
NODE_LAST = N_NODE - (NW - 1) * NODE_CHUNK
NODE_ACC = NODE_LAST

_mesh = plsc.VectorSubcoreMesh(
    core_axis_name="c", subcore_axis_name="s", num_cores=NC, num_subcores=NS)


def _worker_id():
    return lax.axis_index("s") * NC + lax.axis_index("c")


def _zero_rows(acc_v, nrows):
    z = jnp.zeros((LANES,), jnp.float32)

    def zrow(r, carry):
        for c in range(DSL):
            acc_v[r, pl.ds(c * LANES, LANES)] = z
        return carry
    lax.fori_loop(0, nrows, zrow, None)


def _s1_body(bounds, relids, segids, rval, rel_emb, tri_out,
             bnd_v, idx_v, seg_v, val_v, rows_v, acc_v, sem, sem2, sem3, sem4):
    w = _worker_id()

    def chunk(ci, carry):
        j = w + ci * NW

        @pl.when(j < N_SEG_CHUNKS)
        def _():
            pltpu.sync_copy(bounds.at[j], bnd_v)
            bv = bnd_v[...]
            t0 = bv[0]
            t1 = bv[1]
            seg_base = j * SEG_CHUNK
            _zero_rows(acc_v, SEG_CHUNK)
            a8 = (t0 // 8) * 8
            nbat = (t1 - a8 + TB - 1) // TB

            def batch(k, bcarry):
                base = a8 + k * TB
                c1 = pltpu.async_copy(relids.at[pl.ds(base, TB)], idx_v, sem2)
                c2 = pltpu.async_copy(
                    segids.at[pl.ds(base, TB)], seg_v.at[pl.ds(0, TB)], sem3)
                c3 = pltpu.async_copy(rval.at[pl.ds(base, TB)], val_v.at[pl.ds(0, TB)], sem4)
                c1.wait()
                cg = pltpu.async_copy(rel_emb.at[idx_v], rows_v, sem)
                c2.wait()
                c3.wait()
                cg.wait()
                lo = jnp.maximum(t0 - base, 0)
                hi = jnp.minimum(t1 - base, TB)

                def tri(i, tcarry):
                    sc = tcarry[0]
                    ac = tcarry[1:]
                    s_i = seg_v[pl.ds(i, LANES)][0] - seg_base
                    vvb = plsc.load_gather(
                        val_v, [jnp.full((LANES,), i, jnp.int32)])
                    flush = s_i != sc

                    @pl.when(flush & (sc >= 0))
                    def _():
                        for c in range(DSL):
                            sl = pl.ds(c * LANES, LANES)
                            acc_v[sc, sl] = ac[c]

                    new_ac = []
                    for c in range(DSL):
                        sl = pl.ds(c * LANES, LANES)
                        contrib = vvb * rows_v[i, sl]
                        new_ac.append(jnp.where(flush, contrib, ac[c] + contrib))
                    return (s_i,) + tuple(new_ac)
                return lax.fori_loop(lo, hi, tri, bcarry)
            zvv = jnp.zeros((LANES,), jnp.float32)
            tcarry = lax.fori_loop(0, nbat, batch, (-1,) + (zvv,) * DSL)
            sc_f = tcarry[0]
            ac_f = tcarry[1:]

            @pl.when(sc_f >= 0)
            def _():
                for c in range(DSL):
                    sl = pl.ds(c * LANES, LANES)
                    acc_v[sc_f, sl] = ac_f[c]
            pltpu.sync_copy(acc_v, tri_out.at[pl.ds(seg_base, SEG_CHUNK)])
        return carry
    lax.fori_loop(0, SEG_CHUNKS_PER_W, chunk, None)


def _s1_call(bounds, relids, segids, rval, rel_emb):
    f = pl.kernel(
        _s1_body,
        out_type=jax.ShapeDtypeStruct((N_TRI + PAD, N_DIM), jnp.float32),
        mesh=_mesh,
        compiler_params=pltpu.CompilerParams(needs_layout_passes=False),
        scratch_types=[
            pltpu.VMEM((LANES,), jnp.int32),
            pltpu.VMEM((TB,), jnp.int32),
            pltpu.VMEM((TB + LANES,), jnp.int32),
            pltpu.VMEM((TB + LANES,), jnp.float32),
            pltpu.VMEM((TB, N_DIM), jnp.float32),
            pltpu.VMEM((SEG_CHUNK, N_DIM), jnp.float32),
            pltpu.SemaphoreType.DMA,
            pltpu.SemaphoreType.DMA,
            pltpu.SemaphoreType.DMA,
            pltpu.SemaphoreType.DMA,
        ],
    )
    return f(bounds, relids, segids, rval, rel_emb)


def _s3_body(bounds, adj0, adj1, ev, feats, vhat, ha, out, pe_out,
             bnd_v, seg_v, e_v, idx_v, f_rows, v_rows, dot_buf, dots_v,
             acc_v, ha_v, pe_st, sem, sem2, sem3, sem4, sem5):
    w = _worker_id()
    pltpu.sync_copy(bounds.at[w], bnd_v)
    bv = bnd_v[...]
    b0 = bv[0]
    b1 = bv[1]
    n0 = w * NODE_CHUNK
    _zero_rows(acc_v, NODE_ACC)
    io = lax.iota(jnp.int32, LANES)
    pltpu.sync_copy(ha, ha_v)
    one16 = jnp.full((LANES,), 1.0, jnp.float32)

    def pinit(r, carry):
        pe_st[pl.ds(r * LANES, LANES)] = one16
        return carry
    lax.fori_loop(0, (NODE_ACC + LANES - 1) // LANES, pinit, None)
    lane0 = io == 0
    zv = jnp.zeros((LANES,), jnp.float32)

    a8 = (b0 // 8) * 8
    nbat = (b1 - a8 + TB - 1) // TB

    def batch(k, bcarry):
        d_cur = bcarry[0]
        sreg = bcarry[1]
        accs = bcarry[2:]
        base = a8 + k * TB
        c1 = pltpu.async_copy(adj1.at[pl.ds(base, TB)], idx_v, sem2)
        c2 = pltpu.async_copy(
            adj0.at[pl.ds(base, TB)], seg_v.at[pl.ds(0, TB)], sem3)
        c3 = pltpu.async_copy(ev.at[pl.ds(base, TB)], e_v.at[pl.ds(0, TB)], sem4)
        c4 = pltpu.async_copy(vhat.at[pl.ds(base, TB)], v_rows, sem5)
        c1.wait()
        cg = pltpu.async_copy(feats.at[idx_v], f_rows, sem)
        c2.wait()
        c3.wait()
        c4.wait()
        cg.wait()
        lo = jnp.maximum(b0 - base, 0)
        hi = jnp.minimum(b1 - base, TB)

        def dgrp(g, dcarry):
            for u in range(LANES):
                i = g * LANES + u
                p0 = f_rows[i, pl.ds(0, LANES)] * v_rows[i, pl.ds(0, LANES)]
                p1 = f_rows[i, pl.ds(LANES, LANES)] * v_rows[i, pl.ds(LANES, LANES)]
                for c in range(2, DSL, 2):
                    sl = pl.ds(c * LANES, LANES)
                    sl2 = pl.ds((c + 1) * LANES, LANES)
                    p0 = p0 + f_rows[i, sl] * v_rows[i, sl]
                    p1 = p1 + f_rows[i, sl2] * v_rows[i, sl2]
                dot_buf[u, pl.ds(0, LANES)] = p0 + p1
            cols = [plsc.load_gather(dot_buf, [io, jnp.full((LANES,), c, jnp.int32)])
                    for c in range(LANES)]
            while len(cols) > 1:
                cols = [cols[t] + cols[t + 1] for t in range(0, len(cols), 2)]
            dots_v[pl.ds(g * LANES, LANES)] = cols[0]
            return dcarry
        lax.fori_loop(0, TB // LANES, dgrp, None)

        def edge(i, ecarry):
            dc = ecarry[0]
            sr = ecarry[1]
            ac = ecarry[2:]
            d_i = seg_v[pl.ds(i, LANES)][0] - n0
            ibc = jnp.full((LANES,), i, jnp.int32)
            eeb = plsc.load_gather(e_v, [ibc])
            dotb = plsc.load_gather(dots_v, [ibc])
            coefb = dotb * (-2.0 * eeb)
            flush = d_i != dc

            def _do_flush(dc_, sr_, ac_):
                rsv = 1.0 / (sr_ + 1e-16)
                hacc = None
                for c in range(DSL):
                    sl = pl.ds(c * LANES, LANES)
                    t = ac_[c] * rsv
                    fvec = 1.0 - 2.0 / (jnp.exp(t + t) + 1.0)
                    acc_v[dc_, sl] = fvec
                    hp = fvec * ha_v[sl]
                    hacc = hp if hacc is None else hacc + hp
                for kk in (8, 4, 2, 1):
                    dot_buf[0, pl.ds(0, LANES)] = hacc
                    hacc = hacc + plsc.load_gather(
                        dot_buf, [jnp.zeros((LANES,), jnp.int32), io ^ kk])
                plsc.store_scatter(
                    pe_st, [jnp.full((LANES,), dc_, jnp.int32)],
                    jnp.exp(hacc), mask=lane0)

            @pl.when(flush & (dc >= 0))
            def _():
                _do_flush(dc, sr, ac)

            new_ac = []
            for c in range(DSL):
                sl = pl.ds(c * LANES, LANES)
                contrib = eeb * f_rows[i, sl] + coefb * v_rows[i, sl]
                new_ac.append(jnp.where(flush, contrib, ac[c] + contrib))
            sr_new = jnp.where(flush, eeb, sr + eeb)
            return (d_i, sr_new) + tuple(new_ac)
        ecarry = lax.fori_loop(lo, hi, edge, (d_cur, sreg) + accs)
        return ecarry
    fcarry = lax.fori_loop(
        0, nbat, batch, (-1, zv) + (zv,) * DSL)
    d_cur = fcarry[0]
    sreg = fcarry[1]
    accs = fcarry[2:]

    @pl.when(d_cur >= 0)
    def _():
        rsv = 1.0 / (sreg + 1e-16)
        hacc = None
        for c in range(DSL):
            sl = pl.ds(c * LANES, LANES)
            t = accs[c] * rsv
            fvec = 1.0 - 2.0 / (jnp.exp(t + t) + 1.0)
            acc_v[d_cur, sl] = fvec
            hp = fvec * ha_v[sl]
            hacc = hp if hacc is None else hacc + hp
        for kk in (8, 4, 2, 1):
            dot_buf[0, pl.ds(0, LANES)] = hacc
            hacc = hacc + plsc.load_gather(
                dot_buf, [jnp.zeros((LANES,), jnp.int32), io ^ kk])
        plsc.store_scatter(
            pe_st, [jnp.full((LANES,), d_cur, jnp.int32)],
            jnp.exp(hacc), mask=lane0)

    pltpu.sync_copy(acc_v.at[pl.ds(0, NODE_CHUNK)], out.at[pl.ds(n0, NODE_CHUNK)])
    pltpu.sync_copy(pe_st.at[pl.ds(0, NODE_CHUNK)], pe_out.at[pl.ds(n0, NODE_CHUNK)])

    @pl.when(w == NW - 1)
    def _():
        pltpu.sync_copy(
            acc_v.at[pl.ds(NODE_CHUNK, NODE_LAST - NODE_CHUNK)],
            out.at[pl.ds(NW * NODE_CHUNK, NODE_LAST - NODE_CHUNK)])
        pltpu.sync_copy(
            pe_st.at[pl.ds(NODE_CHUNK, NODE_LAST - NODE_CHUNK)],
            pe_out.at[pl.ds(NW * NODE_CHUNK, NODE_LAST - NODE_CHUNK)])


def _s3_call(bounds, adj0, adj1, ev, feats, vhat, ha):
    f = pl.kernel(
        _s3_body,
        out_type=[jax.ShapeDtypeStruct((N_NODE, N_DIM), jnp.float32),
                  jax.ShapeDtypeStruct((N_NODE,), jnp.float32)],
        mesh=_mesh,
        compiler_params=pltpu.CompilerParams(needs_layout_passes=False),
        scratch_types=[
            pltpu.VMEM((LANES,), jnp.int32),
            pltpu.VMEM((TB + LANES,), jnp.int32),
            pltpu.VMEM((TB + LANES,), jnp.float32),
            pltpu.VMEM((TB,), jnp.int32),
            pltpu.VMEM((TB, N_DIM), jnp.float32),
            pltpu.VMEM((TB, N_DIM), jnp.float32),
            pltpu.VMEM((LANES, LANES), jnp.float32),
            pltpu.VMEM((TB + LANES,), jnp.float32),
            pltpu.VMEM((NODE_ACC, N_DIM), jnp.float32),
            pltpu.VMEM((N_DIM,), jnp.float32),
            pltpu.VMEM((NODE_ACC + LANES,), jnp.float32),
            pltpu.SemaphoreType.DMA,
            pltpu.SemaphoreType.DMA,
            pltpu.SemaphoreType.DMA,
            pltpu.SemaphoreType.DMA,
            pltpu.SemaphoreType.DMA,
        ],
    )
    return f(bounds, adj0, adj1, ev, feats, vhat, ha)


def _s4_body(bounds, hi0, hi1, pe, feats, ha, out, pe_out,
             bnd_v, seg_v, idx_v, pe_v, q_buf, f_rows, acc_v, ha_v, pe_st,
             dot_scr, sem, sem2, sem3):
    w = _worker_id()
    pltpu.sync_copy(bounds.at[w], bnd_v)
    bv = bnd_v[...]
    b0 = bv[0]
    b1 = bv[1]
    n0 = w * NODE_CHUNK
    pltpu.sync_copy(pe, pe_v.at[pl.ds(0, N_NODE)])
    _zero_rows(acc_v, NODE_ACC)
    io = lax.iota(jnp.int32, LANES)
    pltpu.sync_copy(ha, ha_v)
    one16 = jnp.full((LANES,), 1.0, jnp.float32)

    def pinit(r, carry):
        pe_st[pl.ds(r * LANES, LANES)] = one16
        return carry
    lax.fori_loop(0, (NODE_ACC + LANES - 1) // LANES, pinit, None)
    lane0 = io == 0

    a8 = (b0 // 8) * 8
    nbat = (b1 - a8 + TB - 1) // TB

    zv = jnp.zeros((LANES,), jnp.float32)

    def batch(k, bcarry):
        d_cur = bcarry[0]
        sreg = bcarry[1]
        accs = bcarry[2:]
        base = a8 + k * TB
        c1 = pltpu.async_copy(hi1.at[pl.ds(base, TB)], idx_v, sem2)
        c2 = pltpu.async_copy(
            hi0.at[pl.ds(base, TB)], seg_v.at[pl.ds(0, TB)], sem3)
        c1.wait()
        cg = pltpu.async_copy(feats.at[idx_v], f_rows, sem)

        def qg(g, qcarry):
            sv = idx_v[pl.ds(g * LANES, LANES)]
            q_buf[pl.ds(g * LANES, LANES)] = plsc.load_gather(pe_v, [sv])
            return qcarry
        lax.fori_loop(0, TB // LANES, qg, None)
        c2.wait()
        cg.wait()
        lo = jnp.maximum(b0 - base, 0)
        hi = jnp.minimum(b1 - base, TB)

        def edge(i, ecarry):
            dc = ecarry[0]
            sr = ecarry[1]
            ac = ecarry[2:]
            d_i = seg_v[pl.ds(i, LANES)][0] - n0
            qb = plsc.load_gather(q_buf, [jnp.full((LANES,), i, jnp.int32)])
            flush = d_i != dc

            @pl.when(flush & (dc >= 0))
            def _():
                rsv = 1.0 / (sr + 1e-16)
                hacc = None
                for c in range(DSL):
                    sl = pl.ds(c * LANES, LANES)
                    t = ac[c] * rsv
                    fvec = 1.0 - 2.0 / (jnp.exp(t + t) + 1.0)
                    acc_v[dc, sl] = fvec
                    hp = fvec * ha_v[sl]
                    hacc = hp if hacc is None else hacc + hp
                for kk in (8, 4, 2, 1):
                    dot_scr[...] = hacc
                    hacc = hacc + plsc.load_gather(dot_scr, [io ^ kk])
                plsc.store_scatter(
                    pe_st, [jnp.full((LANES,), dc, jnp.int32)],
                    jnp.exp(hacc), mask=lane0)

            new_ac = []
            for c in range(DSL):
                sl = pl.ds(c * LANES, LANES)
                contrib = qb * f_rows[i, sl]
                new_ac.append(jnp.where(flush, contrib, ac[c] + contrib))
            sr_new = jnp.where(flush, qb, sr + qb)
            return (d_i, sr_new) + tuple(new_ac)
        ecarry = lax.fori_loop(lo, hi, edge, (d_cur, sreg) + accs)
        return ecarry
    fcarry = lax.fori_loop(0, nbat, batch, (-1, zv) + (zv,) * DSL)
    d_cur = fcarry[0]
    sreg = fcarry[1]
    accs = fcarry[2:]

    @pl.when(d_cur >= 0)
    def _():
        rsv = 1.0 / (sreg + 1e-16)
        hacc = None
        for c in range(DSL):
            sl = pl.ds(c * LANES, LANES)
            t = accs[c] * rsv
            fvec = 1.0 - 2.0 / (jnp.exp(t + t) + 1.0)
            acc_v[d_cur, sl] = fvec
            hp = fvec * ha_v[sl]
            hacc = hp if hacc is None else hacc + hp
        for kk in (8, 4, 2, 1):
            dot_scr[...] = hacc
            hacc = hacc + plsc.load_gather(dot_scr, [io ^ kk])
        plsc.store_scatter(
            pe_st, [jnp.full((LANES,), d_cur, jnp.int32)],
            jnp.exp(hacc), mask=lane0)

    pltpu.sync_copy(acc_v.at[pl.ds(0, NODE_CHUNK)], out.at[pl.ds(n0, NODE_CHUNK)])
    pltpu.sync_copy(pe_st.at[pl.ds(0, NODE_CHUNK)], pe_out.at[pl.ds(n0, NODE_CHUNK)])

    @pl.when(w == NW - 1)
    def _():
        pltpu.sync_copy(
            acc_v.at[pl.ds(NODE_CHUNK, NODE_LAST - NODE_CHUNK)],
            out.at[pl.ds(NW * NODE_CHUNK, NODE_LAST - NODE_CHUNK)])
        pltpu.sync_copy(
            pe_st.at[pl.ds(NODE_CHUNK, NODE_LAST - NODE_CHUNK)],
            pe_out.at[pl.ds(NW * NODE_CHUNK, NODE_LAST - NODE_CHUNK)])


def _s4_call(bounds, hi0, hi1, pe, feats, ha):
    f = pl.kernel(
        _s4_body,
        out_type=[jax.ShapeDtypeStruct((N_NODE, N_DIM), jnp.float32),
                  jax.ShapeDtypeStruct((N_NODE,), jnp.float32)],
        mesh=_mesh,
        compiler_params=pltpu.CompilerParams(needs_layout_passes=False),
        scratch_types=[
            pltpu.VMEM((LANES,), jnp.int32),
            pltpu.VMEM((TB + LANES,), jnp.int32),
            pltpu.VMEM((TB,), jnp.int32),
            pltpu.VMEM((N_NODE + LANES,), jnp.float32),
            pltpu.VMEM((TB + LANES,), jnp.float32),
            pltpu.VMEM((TB, N_DIM), jnp.float32),
            pltpu.VMEM((NODE_ACC, N_DIM), jnp.float32),
            pltpu.VMEM((N_DIM,), jnp.float32),
            pltpu.VMEM((NODE_ACC + LANES,), jnp.float32),
            pltpu.VMEM((LANES,), jnp.float32),
            pltpu.SemaphoreType.DMA,
            pltpu.SemaphoreType.DMA,
            pltpu.SemaphoreType.DMA,
        ],
    )
    return f(bounds, hi0, hi1, pe, feats, ha)


def _tanh_body(x_ref, o_ref):
    o_ref[...] = jnp.tanh(x_ref[...])


def _tanh_call(x):
    return pl.pallas_call(
        _tanh_body,
        out_shape=jax.ShapeDtypeStruct(x.shape, jnp.float32),
    )(x)


def _t1_body(tri_ref, k01_ref, v_ref, e_ref):
    x = tri_ref[...]
    ss = jnp.sum(x * x, axis=1, keepdims=True)
    v = x / jnp.maximum(jnp.sqrt(ss), 1e-12)
    v_ref[...] = v
    e_ref[...] = jnp.exp(v @ k01_ref[...])


def _t1_call(tri, k01):
    n = tri.shape[0]
    br = 1024
    return pl.pallas_call(
        _t1_body,
        grid=(n // br,),
        in_specs=[pl.BlockSpec((br, N_DIM), lambda i: (i, 0)),
                  pl.BlockSpec((N_DIM, 2), lambda i: (0, 0))],
        out_specs=[pl.BlockSpec((br, N_DIM), lambda i: (i, 0)),
                   pl.BlockSpec((br, 2), lambda i: (i, 0))],
        out_shape=[jax.ShapeDtypeStruct((n, N_DIM), jnp.float32),
                   jax.ShapeDtypeStruct((n, 2), jnp.float32)],
    )(tri, k01)


def kernel(features, rel_emb, adj, r_index, r_val, high_nei,
           attn_k0, attn_k1, high_att0, high_att1):
    r0, r1 = r_index[0], r_index[1]
    a0, a1 = adj[0], adj[1]
    h0, h1 = high_nei[0], high_nei[1]

    seg_grid = jnp.arange(N_SEG_CHUNKS + 1, dtype=jnp.int32) * SEG_CHUNK
    sb = jnp.searchsorted(r0, seg_grid, side="left").astype(jnp.int32)
    s1_bounds = jnp.zeros((NW * SEG_CHUNKS_PER_W, LANES), jnp.int32)
    s1_bounds = s1_bounds.at[:N_SEG_CHUNKS, 0].set(sb[:-1])
    s1_bounds = s1_bounds.at[:N_SEG_CHUNKS, 1].set(sb[1:])

    node_grid = jnp.concatenate([
        jnp.arange(NW, dtype=jnp.int32) * NODE_CHUNK,
        jnp.array([N_NODE], jnp.int32)])
    ab = jnp.searchsorted(a0, node_grid, side="left").astype(jnp.int32)
    s3_bounds = jnp.zeros((NW, LANES), jnp.int32)
    s3_bounds = s3_bounds.at[:, 0].set(ab[:-1]).at[:, 1].set(ab[1:])
    hb = jnp.searchsorted(h0, node_grid, side="left").astype(jnp.int32)
    s4_bounds = jnp.zeros((NW, LANES), jnp.int32)
    s4_bounds = s4_bounds.at[:, 0].set(hb[:-1]).at[:, 1].set(hb[1:])

    pad1 = lambda x: jnp.pad(x, (0, PAD))
    r0p, r1p, rvp = pad1(r0), pad1(r1), pad1(r_val)
    a0p, a1p = pad1(a0), pad1(a1)
    h0p, h1p = pad1(h0), pad1(h1)

    tri = _s1_call(s1_bounds, r1p, r0p, rvp, rel_emb)
    k01 = jnp.concatenate([attn_k0, attn_k1], axis=1)
    vhat, e01 = _t1_call(tri, k01)
    e0 = jnp.asarray(e01[:, 0])
    e1 = jnp.asarray(e01[:, 1])

    ha0 = jnp.squeeze(high_att0, -1)
    ha1 = jnp.squeeze(high_att1, -1)
    f0 = _tanh_call(features)
    f1, _ = _s3_call(s3_bounds, a0p, a1p, e0, f0, vhat, ha0)
    f2, pe0 = _s3_call(s3_bounds, a0p, a1p, e1, f1, vhat, ha0)
    f3, pe1 = _s4_call(s4_bounds, h0p, h1p, pe0, f2, ha1)
    f4, _ = _s4_call(s4_bounds, h0p, h1p, pe1, f3, ha1)
    return jnp.concatenate([f0, f1, f2, f3, f4], axis=-1)


# speedup vs baseline: 6.6681x; 1.0387x over previous
import jax
import jax.numpy as jnp
from jax import lax
from jax.experimental import pallas as pl
from jax.experimental.pallas import tpu as pltpu
from jax.experimental.pallas import tpu_sc as plsc

N_NODE = 10000
N_DIM = 128
N_EDGE = 320000
N_TRI = 320000
LANES = 16
DSL = N_DIM // LANES

NC, NS = 2, 16
NW = NC * NS

TB = 256
PAD = 512
SEG_CHUNK = 512
N_SEG_CHUNKS = N_TRI // SEG_CHUNK
SEG_CHUNKS_PER_W = -(-N_SEG_CHUNKS // NW)
NODE_CHUNK = 312
NODE_LAST = N_NODE - (NW - 1) * NODE_CHUNK
NODE_ACC = NODE_LAST

_mesh = plsc.VectorSubcoreMesh(
    core_axis_name="c", subcore_axis_name="s", num_cores=NC, num_subcores=NS)


def _worker_id():
    return lax.axis_index("s") * NC + lax.axis_index("c")


def _zero_rows(acc_v, nrows):
    z = jnp.zeros((LANES,), jnp.float32)

    def zrow(r, carry):
        for c in range(DSL):
            acc_v[r, pl.ds(c * LANES, LANES)] = z
        return carry
    lax.fori_loop(0, nrows, zrow, None)


def _s1_body(bounds, relids, segids, rval, rel_emb, tri_out,
             bnd_v, idx_v, seg_v, val_v, rows_v, acc_v, sem, sem2, sem3, sem4):
    w = _worker_id()

    def chunk(ci, carry):
        j = w + ci * NW

        @pl.when(j < N_SEG_CHUNKS)
        def _():
            pltpu.sync_copy(bounds.at[j], bnd_v)
            bv = bnd_v[...]
            t0 = bv[0]
            t1 = bv[1]
            seg_base = j * SEG_CHUNK
            _zero_rows(acc_v, SEG_CHUNK)
            a8 = (t0 // 8) * 8
            nbat = (t1 - a8 + TB - 1) // TB

            def batch(k, bcarry):
                base = a8 + k * TB
                c1 = pltpu.async_copy(relids.at[pl.ds(base, TB)], idx_v, sem2)
                c2 = pltpu.async_copy(
                    segids.at[pl.ds(base, TB)], seg_v.at[pl.ds(0, TB)], sem3)
                c3 = pltpu.async_copy(rval.at[pl.ds(base, TB)], val_v.at[pl.ds(0, TB)], sem4)
                c1.wait()
                cg = pltpu.async_copy(rel_emb.at[idx_v], rows_v, sem)
                c2.wait()
                c3.wait()
                cg.wait()
                lo = jnp.maximum(t0 - base, 0)
                hi = jnp.minimum(t1 - base, TB)

                def tri(i, tcarry):
                    sc = tcarry[0]
                    ac = tcarry[1:]
                    s_i = seg_v[pl.ds(i, LANES)][0] - seg_base
                    vvb = plsc.load_gather(
                        val_v, [jnp.full((LANES,), i, jnp.int32)])
                    flush = s_i != sc

                    @pl.when(flush & (sc >= 0))
                    def _():
                        for c in range(DSL):
                            sl = pl.ds(c * LANES, LANES)
                            acc_v[sc, sl] = ac[c]

                    new_ac = []
                    for c in range(DSL):
                        sl = pl.ds(c * LANES, LANES)
                        contrib = vvb * rows_v[i, sl]
                        new_ac.append(jnp.where(flush, contrib, ac[c] + contrib))
                    return (s_i,) + tuple(new_ac)
                return lax.fori_loop(lo, hi, tri, bcarry)
            zvv = jnp.zeros((LANES,), jnp.float32)
            tcarry = lax.fori_loop(0, nbat, batch, (-1,) + (zvv,) * DSL)
            sc_f = tcarry[0]
            ac_f = tcarry[1:]

            @pl.when(sc_f >= 0)
            def _():
                for c in range(DSL):
                    sl = pl.ds(c * LANES, LANES)
                    acc_v[sc_f, sl] = ac_f[c]
            pltpu.sync_copy(acc_v, tri_out.at[pl.ds(seg_base, SEG_CHUNK)])
        return carry
    lax.fori_loop(0, SEG_CHUNKS_PER_W, chunk, None)


def _s1_call(bounds, relids, segids, rval, rel_emb):
    f = pl.kernel(
        _s1_body,
        out_type=jax.ShapeDtypeStruct((N_TRI + PAD, N_DIM), jnp.float32),
        mesh=_mesh,
        compiler_params=pltpu.CompilerParams(needs_layout_passes=False),
        scratch_types=[
            pltpu.VMEM((LANES,), jnp.int32),
            pltpu.VMEM((TB,), jnp.int32),
            pltpu.VMEM((TB + LANES,), jnp.int32),
            pltpu.VMEM((TB + LANES,), jnp.float32),
            pltpu.VMEM((TB, N_DIM), jnp.float32),
            pltpu.VMEM((SEG_CHUNK, N_DIM), jnp.float32),
            pltpu.SemaphoreType.DMA,
            pltpu.SemaphoreType.DMA,
            pltpu.SemaphoreType.DMA,
            pltpu.SemaphoreType.DMA,
        ],
    )
    return f(bounds, relids, segids, rval, rel_emb)


def _s3_body(bounds, adj0, adj1, ev, feats, vhat, ha, out, pe_out,
             bnd_v, seg_v, e_v, idx_v, f_rows, v_rows, dot_buf, dots_v,
             acc_v, ha_v, pe_st, sem, sem2, sem3, sem4, sem5):
    w = _worker_id()
    pltpu.sync_copy(bounds.at[w], bnd_v)
    bv = bnd_v[...]
    b0 = bv[0]
    b1 = bv[1]
    n0 = w * NODE_CHUNK
    _zero_rows(acc_v, NODE_ACC)
    io = lax.iota(jnp.int32, LANES)
    pltpu.sync_copy(ha, ha_v)
    one16 = jnp.full((LANES,), 1.0, jnp.float32)

    def pinit(r, carry):
        pe_st[pl.ds(r * LANES, LANES)] = one16
        return carry
    lax.fori_loop(0, (NODE_ACC + LANES - 1) // LANES, pinit, None)
    lane0 = io == 0
    zv = jnp.zeros((LANES,), jnp.float32)

    a8 = (b0 // 8) * 8
    nbat = (b1 - a8 + TB - 1) // TB

    def batch(k, bcarry):
        d_cur = bcarry[0]
        sreg = bcarry[1]
        accs = bcarry[2:]
        base = a8 + k * TB
        c1 = pltpu.async_copy(adj1.at[pl.ds(base, TB)], idx_v, sem2)
        c2 = pltpu.async_copy(
            adj0.at[pl.ds(base, TB)], seg_v.at[pl.ds(0, TB)], sem3)
        c3 = pltpu.async_copy(ev.at[pl.ds(base, TB)], e_v.at[pl.ds(0, TB)], sem4)
        c4 = pltpu.async_copy(vhat.at[pl.ds(base, TB)], v_rows, sem5)
        c1.wait()
        cg = pltpu.async_copy(feats.at[idx_v], f_rows, sem)
        c2.wait()
        c3.wait()
        c4.wait()
        cg.wait()
        lo = jnp.maximum(b0 - base, 0)
        hi = jnp.minimum(b1 - base, TB)

        def dgrp(g, dcarry):
            for u in range(LANES):
                i = g * LANES + u
                p0 = f_rows[i, pl.ds(0, LANES)] * v_rows[i, pl.ds(0, LANES)]
                p1 = f_rows[i, pl.ds(LANES, LANES)] * v_rows[i, pl.ds(LANES, LANES)]
                for c in range(2, DSL, 2):
                    sl = pl.ds(c * LANES, LANES)
                    sl2 = pl.ds((c + 1) * LANES, LANES)
                    p0 = p0 + f_rows[i, sl] * v_rows[i, sl]
                    p1 = p1 + f_rows[i, sl2] * v_rows[i, sl2]
                dot_buf[u, pl.ds(0, LANES)] = p0 + p1
            cols = [plsc.load_gather(dot_buf, [io, jnp.full((LANES,), c, jnp.int32)])
                    for c in range(LANES)]
            while len(cols) > 1:
                cols = [cols[t] + cols[t + 1] for t in range(0, len(cols), 2)]
            dots_v[pl.ds(g * LANES, LANES)] = cols[0]
            return dcarry
        lax.fori_loop(0, TB // LANES, dgrp, None)

        def edge(i, ecarry):
            dc = ecarry[0]
            sr = ecarry[1]
            ac = ecarry[2:]
            d_i = seg_v[pl.ds(i, LANES)][0] - n0
            ibc = jnp.full((LANES,), i, jnp.int32)
            eeb = plsc.load_gather(e_v, [ibc])
            dotb = plsc.load_gather(dots_v, [ibc])
            coefb = dotb * (-2.0 * eeb)
            flush = d_i != dc

            def _do_flush(dc_, sr_, ac_):
                rsv = 1.0 / (sr_ + 1e-16)
                hacc = None
                for c in range(DSL):
                    sl = pl.ds(c * LANES, LANES)
                    t = ac_[c] * rsv
                    fvec = 1.0 - 2.0 / (jnp.exp(t + t) + 1.0)
                    acc_v[dc_, sl] = fvec
                    hp = fvec * ha_v[sl]
                    hacc = hp if hacc is None else hacc + hp
                for kk in (8, 4, 2, 1):
                    dot_buf[0, pl.ds(0, LANES)] = hacc
                    hacc = hacc + plsc.load_gather(
                        dot_buf, [jnp.zeros((LANES,), jnp.int32), io ^ kk])
                plsc.store_scatter(
                    pe_st, [jnp.full((LANES,), dc_, jnp.int32)],
                    jnp.exp(hacc), mask=lane0)

            @pl.when(flush & (dc >= 0))
            def _():
                _do_flush(dc, sr, ac)

            new_ac = []
            for c in range(DSL):
                sl = pl.ds(c * LANES, LANES)
                contrib = eeb * f_rows[i, sl] + coefb * v_rows[i, sl]
                new_ac.append(jnp.where(flush, contrib, ac[c] + contrib))
            sr_new = jnp.where(flush, eeb, sr + eeb)
            return (d_i, sr_new) + tuple(new_ac)
        ecarry = lax.fori_loop(lo, hi, edge, (d_cur, sreg) + accs)
        return ecarry
    fcarry = lax.fori_loop(
        0, nbat, batch, (-1, zv) + (zv,) * DSL)
    d_cur = fcarry[0]
    sreg = fcarry[1]
    accs = fcarry[2:]

    @pl.when(d_cur >= 0)
    def _():
        rsv = 1.0 / (sreg + 1e-16)
        hacc = None
        for c in range(DSL):
            sl = pl.ds(c * LANES, LANES)
            t = accs[c] * rsv
            fvec = 1.0 - 2.0 / (jnp.exp(t + t) + 1.0)
            acc_v[d_cur, sl] = fvec
            hp = fvec * ha_v[sl]
            hacc = hp if hacc is None else hacc + hp
        for kk in (8, 4, 2, 1):
            dot_buf[0, pl.ds(0, LANES)] = hacc
            hacc = hacc + plsc.load_gather(
                dot_buf, [jnp.zeros((LANES,), jnp.int32), io ^ kk])
        plsc.store_scatter(
            pe_st, [jnp.full((LANES,), d_cur, jnp.int32)],
            jnp.exp(hacc), mask=lane0)

    pltpu.sync_copy(acc_v.at[pl.ds(0, NODE_CHUNK)], out.at[pl.ds(n0, NODE_CHUNK)])
    pltpu.sync_copy(pe_st.at[pl.ds(0, NODE_CHUNK)], pe_out.at[pl.ds(n0, NODE_CHUNK)])

    @pl.when(w == NW - 1)
    def _():
        pltpu.sync_copy(
            acc_v.at[pl.ds(NODE_CHUNK, NODE_LAST - NODE_CHUNK)],
            out.at[pl.ds(NW * NODE_CHUNK, NODE_LAST - NODE_CHUNK)])
        pltpu.sync_copy(
            pe_st.at[pl.ds(NODE_CHUNK, NODE_LAST - NODE_CHUNK)],
            pe_out.at[pl.ds(NW * NODE_CHUNK, NODE_LAST - NODE_CHUNK)])


def _s3_call(bounds, adj0, adj1, ev, feats, vhat, ha):
    f = pl.kernel(
        _s3_body,
        out_type=[jax.ShapeDtypeStruct((N_NODE, N_DIM), jnp.float32),
                  jax.ShapeDtypeStruct((N_NODE,), jnp.float32)],
        mesh=_mesh,
        compiler_params=pltpu.CompilerParams(needs_layout_passes=False),
        scratch_types=[
            pltpu.VMEM((LANES,), jnp.int32),
            pltpu.VMEM((TB + LANES,), jnp.int32),
            pltpu.VMEM((TB + LANES,), jnp.float32),
            pltpu.VMEM((TB,), jnp.int32),
            pltpu.VMEM((TB, N_DIM), jnp.float32),
            pltpu.VMEM((TB, N_DIM), jnp.float32),
            pltpu.VMEM((LANES, LANES), jnp.float32),
            pltpu.VMEM((TB + LANES,), jnp.float32),
            pltpu.VMEM((NODE_ACC, N_DIM), jnp.float32),
            pltpu.VMEM((N_DIM,), jnp.float32),
            pltpu.VMEM((NODE_ACC + LANES,), jnp.float32),
            pltpu.SemaphoreType.DMA,
            pltpu.SemaphoreType.DMA,
            pltpu.SemaphoreType.DMA,
            pltpu.SemaphoreType.DMA,
            pltpu.SemaphoreType.DMA,
        ],
    )
    return f(bounds, adj0, adj1, ev, feats, vhat, ha)


def _s4_body(bounds, hi0, hi1, pe, feats, ha, out, pe_out,
             bnd_v, seg_v, idx_v, pe_v, q_buf, f_rows, acc_v, ha_v, pe_st,
             dot_scr, sem, sem2, sem3):
    w = _worker_id()
    pltpu.sync_copy(bounds.at[w], bnd_v)
    bv = bnd_v[...]
    b0 = bv[0]
    b1 = bv[1]
    n0 = w * NODE_CHUNK
    pltpu.sync_copy(pe, pe_v.at[pl.ds(0, N_NODE)])
    _zero_rows(acc_v, NODE_ACC)
    io = lax.iota(jnp.int32, LANES)
    pltpu.sync_copy(ha, ha_v)
    one16 = jnp.full((LANES,), 1.0, jnp.float32)

    def pinit(r, carry):
        pe_st[pl.ds(r * LANES, LANES)] = one16
        return carry
    lax.fori_loop(0, (NODE_ACC + LANES - 1) // LANES, pinit, None)
    lane0 = io == 0

    a8 = (b0 // 8) * 8
    nbat = (b1 - a8 + TB - 1) // TB

    zv = jnp.zeros((LANES,), jnp.float32)

    def batch(k, bcarry):
        d_cur = bcarry[0]
        sreg = bcarry[1]
        accs = bcarry[2:]
        base = a8 + k * TB
        c1 = pltpu.async_copy(hi1.at[pl.ds(base, TB)], idx_v, sem2)
        c2 = pltpu.async_copy(
            hi0.at[pl.ds(base, TB)], seg_v.at[pl.ds(0, TB)], sem3)
        c1.wait()
        cg = pltpu.async_copy(feats.at[idx_v], f_rows, sem)

        def qg(g, qcarry):
            sv = idx_v[pl.ds(g * LANES, LANES)]
            q_buf[pl.ds(g * LANES, LANES)] = plsc.load_gather(pe_v, [sv])
            return qcarry
        lax.fori_loop(0, TB // LANES, qg, None)
        c2.wait()
        cg.wait()
        lo = jnp.maximum(b0 - base, 0)
        hi = jnp.minimum(b1 - base, TB)

        def edge(i, ecarry):
            dc = ecarry[0]
            sr = ecarry[1]
            ac = ecarry[2:]
            d_i = seg_v[pl.ds(i, LANES)][0] - n0
            qb = plsc.load_gather(q_buf, [jnp.full((LANES,), i, jnp.int32)])
            flush = d_i != dc

            @pl.when(flush & (dc >= 0))
            def _():
                rsv = 1.0 / (sr + 1e-16)
                hacc = None
                for c in range(DSL):
                    sl = pl.ds(c * LANES, LANES)
                    t = ac[c] * rsv
                    fvec = 1.0 - 2.0 / (jnp.exp(t + t) + 1.0)
                    acc_v[dc, sl] = fvec
                    hp = fvec * ha_v[sl]
                    hacc = hp if hacc is None else hacc + hp
                for kk in (8, 4, 2, 1):
                    dot_scr[...] = hacc
                    hacc = hacc + plsc.load_gather(dot_scr, [io ^ kk])
                plsc.store_scatter(
                    pe_st, [jnp.full((LANES,), dc, jnp.int32)],
                    jnp.exp(hacc), mask=lane0)

            new_ac = []
            for c in range(DSL):
                sl = pl.ds(c * LANES, LANES)
                contrib = qb * f_rows[i, sl]
                new_ac.append(jnp.where(flush, contrib, ac[c] + contrib))
            sr_new = jnp.where(flush, qb, sr + qb)
            return (d_i, sr_new) + tuple(new_ac)
        ecarry = lax.fori_loop(lo, hi, edge, (d_cur, sreg) + accs)
        return ecarry
    fcarry = lax.fori_loop(0, nbat, batch, (-1, zv) + (zv,) * DSL)
    d_cur = fcarry[0]
    sreg = fcarry[1]
    accs = fcarry[2:]

    @pl.when(d_cur >= 0)
    def _():
        rsv = 1.0 / (sreg + 1e-16)
        hacc = None
        for c in range(DSL):
            sl = pl.ds(c * LANES, LANES)
            t = accs[c] * rsv
            fvec = 1.0 - 2.0 / (jnp.exp(t + t) + 1.0)
            acc_v[d_cur, sl] = fvec
            hp = fvec * ha_v[sl]
            hacc = hp if hacc is None else hacc + hp
        for kk in (8, 4, 2, 1):
            dot_scr[...] = hacc
            hacc = hacc + plsc.load_gather(dot_scr, [io ^ kk])
        plsc.store_scatter(
            pe_st, [jnp.full((LANES,), d_cur, jnp.int32)],
            jnp.exp(hacc), mask=lane0)

    pltpu.sync_copy(acc_v.at[pl.ds(0, NODE_CHUNK)], out.at[pl.ds(n0, NODE_CHUNK)])
    pltpu.sync_copy(pe_st.at[pl.ds(0, NODE_CHUNK)], pe_out.at[pl.ds(n0, NODE_CHUNK)])

    @pl.when(w == NW - 1)
    def _():
        pltpu.sync_copy(
            acc_v.at[pl.ds(NODE_CHUNK, NODE_LAST - NODE_CHUNK)],
            out.at[pl.ds(NW * NODE_CHUNK, NODE_LAST - NODE_CHUNK)])
        pltpu.sync_copy(
            pe_st.at[pl.ds(NODE_CHUNK, NODE_LAST - NODE_CHUNK)],
            pe_out.at[pl.ds(NW * NODE_CHUNK, NODE_LAST - NODE_CHUNK)])


def _s4_call(bounds, hi0, hi1, pe, feats, ha):
    f = pl.kernel(
        _s4_body,
        out_type=[jax.ShapeDtypeStruct((N_NODE, N_DIM), jnp.float32),
                  jax.ShapeDtypeStruct((N_NODE,), jnp.float32)],
        mesh=_mesh,
        compiler_params=pltpu.CompilerParams(needs_layout_passes=False),
        scratch_types=[
            pltpu.VMEM((LANES,), jnp.int32),
            pltpu.VMEM((TB + LANES,), jnp.int32),
            pltpu.VMEM((TB,), jnp.int32),
            pltpu.VMEM((N_NODE + LANES,), jnp.float32),
            pltpu.VMEM((TB + LANES,), jnp.float32),
            pltpu.VMEM((TB, N_DIM), jnp.float32),
            pltpu.VMEM((NODE_ACC, N_DIM), jnp.float32),
            pltpu.VMEM((N_DIM,), jnp.float32),
            pltpu.VMEM((NODE_ACC + LANES,), jnp.float32),
            pltpu.VMEM((LANES,), jnp.float32),
            pltpu.SemaphoreType.DMA,
            pltpu.SemaphoreType.DMA,
            pltpu.SemaphoreType.DMA,
        ],
    )
    return f(bounds, hi0, hi1, pe, feats, ha)


def _tanh_body(x_ref, o_ref):
    o_ref[...] = jnp.tanh(x_ref[...])


def _tanh_call(x):
    return pl.pallas_call(
        _tanh_body,
        out_shape=jax.ShapeDtypeStruct(x.shape, jnp.float32),
    )(x)


def _t1_body(tri_ref, k01_ref, v_ref, e_ref):
    x = tri_ref[...]
    ss = jnp.sum(x * x, axis=1, keepdims=True)
    v = x / jnp.maximum(jnp.sqrt(ss), 1e-12)
    v_ref[...] = v
    e_ref[...] = jnp.exp(v @ k01_ref[...])


def _t1_call(tri, k01):
    n = tri.shape[0]
    br = 1024
    return pl.pallas_call(
        _t1_body,
        grid=(n // br,),
        in_specs=[pl.BlockSpec((br, N_DIM), lambda i: (i, 0)),
                  pl.BlockSpec((N_DIM, 2), lambda i: (0, 0))],
        out_specs=[pl.BlockSpec((br, N_DIM), lambda i: (i, 0)),
                   pl.BlockSpec((br, 2), lambda i: (i, 0))],
        out_shape=[jax.ShapeDtypeStruct((n, N_DIM), jnp.float32),
                   jax.ShapeDtypeStruct((n, 2), jnp.float32)],
    )(tri, k01)


def kernel(features, rel_emb, adj, r_index, r_val, high_nei,
           attn_k0, attn_k1, high_att0, high_att1):
    r0, r1 = r_index[0], r_index[1]
    a0, a1 = adj[0], adj[1]
    h0, h1 = high_nei[0], high_nei[1]

    seg_grid = jnp.arange(N_SEG_CHUNKS + 1, dtype=jnp.int32) * SEG_CHUNK
    sb = jnp.searchsorted(r0, seg_grid, side="left").astype(jnp.int32)
    s1_bounds = jnp.zeros((NW * SEG_CHUNKS_PER_W, LANES), jnp.int32)
    s1_bounds = s1_bounds.at[:N_SEG_CHUNKS, 0].set(sb[:-1])
    s1_bounds = s1_bounds.at[:N_SEG_CHUNKS, 1].set(sb[1:])

    node_grid = jnp.concatenate([
        jnp.arange(NW, dtype=jnp.int32) * NODE_CHUNK,
        jnp.array([N_NODE], jnp.int32)])
    ab = jnp.searchsorted(a0, node_grid, side="left").astype(jnp.int32)
    s3_bounds = jnp.zeros((NW, LANES), jnp.int32)
    s3_bounds = s3_bounds.at[:, 0].set(ab[:-1]).at[:, 1].set(ab[1:])
    hb = jnp.searchsorted(h0, node_grid, side="left").astype(jnp.int32)
    s4_bounds = jnp.zeros((NW, LANES), jnp.int32)
    s4_bounds = s4_bounds.at[:, 0].set(hb[:-1]).at[:, 1].set(hb[1:])

    pad1 = lambda x: jnp.pad(x, (0, PAD))
    r0p, r1p, rvp = pad1(r0), pad1(r1), pad1(r_val)
    a0p, a1p = pad1(a0), pad1(a1)
    h0p, h1p = pad1(h0), pad1(h1)

    tri = _s1_call(s1_bounds, r1p, r0p, rvp, rel_emb)
    k01 = jnp.concatenate([attn_k0, attn_k1], axis=1)
    vhat, e01 = _t1_call(tri, k01)
    e0 = jnp.asarray(e01[:, 0])
    e1 = jnp.asarray(e01[:, 1])

    ha0 = jnp.squeeze(high_att0, -1)
    ha1 = jnp.squeeze(high_att1, -1)
    f0 = _tanh_call(features)
    f1, _ = _s3_call(s3_bounds, a0p, a1p, e0, f0, vhat, ha0)
    f2, pe0 = _s3_call(s3_bounds, a0p, a1p, e1, f1, vhat, ha0)
    f3, pe1 = _s4_call(s4_bounds, h0p, h1p, pe0, f2, ha1)
    f4, _ = _s4_call(s4_bounds, h0p, h1p, pe1, f3, ha1)
    return jnp.concatenate([f0, f1, f2, f3, f4], axis=-1)


# speedup vs baseline: 6.6784x; 1.0016x over previous
import jax
import jax.numpy as jnp
from jax import lax
from jax.experimental import pallas as pl
from jax.experimental.pallas import tpu as pltpu
from jax.experimental.pallas import tpu_sc as plsc

N_NODE = 10000
N_DIM = 128
N_EDGE = 320000
N_TRI = 320000
LANES = 16
DSL = N_DIM // LANES

NC, NS = 2, 16
NW = NC * NS

TB = 256
PAD = 512
SEG_CHUNK = 512
N_SEG_CHUNKS = N_TRI // SEG_CHUNK
SEG_CHUNKS_PER_W = -(-N_SEG_CHUNKS // NW)
NODE_CHUNK = 312
NODE_LAST = N_NODE - (NW - 1) * NODE_CHUNK
NODE_ACC = NODE_LAST

import functools


@functools.cache
def _mesh_():
    return plsc.VectorSubcoreMesh(
        core_axis_name="c", subcore_axis_name="s",
        num_cores=NC, num_subcores=NS)


def _worker_id():
    return lax.axis_index("s") * NC + lax.axis_index("c")


def _zero_rows(acc_v, nrows):
    z = jnp.zeros((LANES,), jnp.float32)

    def zrow(r, carry):
        for c in range(DSL):
            acc_v[r, pl.ds(c * LANES, LANES)] = z
        return carry
    lax.fori_loop(0, nrows, zrow, None)


def _s1_body(bounds, relids, segids, rval, rel_emb, tri_out,
             bnd_v, idx_v, seg_v, val_v, rows_v, acc_v, sem, sem2, sem3, sem4):
    w = _worker_id()

    def chunk(ci, carry):
        j = w + ci * NW

        @pl.when(j < N_SEG_CHUNKS)
        def _():
            pltpu.sync_copy(bounds.at[j], bnd_v)
            bv = bnd_v[...]
            t0 = bv[0]
            t1 = bv[1]
            seg_base = j * SEG_CHUNK
            _zero_rows(acc_v, SEG_CHUNK)
            a8 = (t0 // 8) * 8
            nbat = (t1 - a8 + TB - 1) // TB

            def batch(k, bcarry):
                base = a8 + k * TB
                c1 = pltpu.async_copy(relids.at[pl.ds(base, TB)], idx_v, sem2)
                c2 = pltpu.async_copy(
                    segids.at[pl.ds(base, TB)], seg_v.at[pl.ds(0, TB)], sem3)
                c3 = pltpu.async_copy(rval.at[pl.ds(base, TB)], val_v.at[pl.ds(0, TB)], sem4)
                c1.wait()
                cg = pltpu.async_copy(rel_emb.at[idx_v], rows_v, sem)
                c2.wait()
                c3.wait()
                cg.wait()
                lo = jnp.maximum(t0 - base, 0)
                hi = jnp.minimum(t1 - base, TB)

                def tri(i, tcarry):
                    sc = tcarry[0]
                    ac = tcarry[1:]
                    s_i = seg_v[pl.ds(i, LANES)][0] - seg_base
                    vvb = plsc.load_gather(
                        val_v, [jnp.full((LANES,), i, jnp.int32)])
                    flush = s_i != sc

                    @pl.when(flush & (sc >= 0))
                    def _():
                        for c in range(DSL):
                            sl = pl.ds(c * LANES, LANES)
                            acc_v[sc, sl] = ac[c]

                    new_ac = []
                    for c in range(DSL):
                        sl = pl.ds(c * LANES, LANES)
                        contrib = vvb * rows_v[i, sl]
                        new_ac.append(jnp.where(flush, contrib, ac[c] + contrib))
                    return (s_i,) + tuple(new_ac)
                return lax.fori_loop(lo, hi, tri, bcarry)
            zvv = jnp.zeros((LANES,), jnp.float32)
            tcarry = lax.fori_loop(0, nbat, batch, (-1,) + (zvv,) * DSL)
            sc_f = tcarry[0]
            ac_f = tcarry[1:]

            @pl.when(sc_f >= 0)
            def _():
                for c in range(DSL):
                    sl = pl.ds(c * LANES, LANES)
                    acc_v[sc_f, sl] = ac_f[c]
            pltpu.sync_copy(acc_v, tri_out.at[pl.ds(seg_base, SEG_CHUNK)])
        return carry
    lax.fori_loop(0, SEG_CHUNKS_PER_W, chunk, None)


def _s1_call(bounds, relids, segids, rval, rel_emb):
    f = pl.kernel(
        _s1_body,
        out_type=jax.ShapeDtypeStruct((N_TRI + PAD, N_DIM), jnp.float32),
        mesh=_mesh_(),
        compiler_params=pltpu.CompilerParams(needs_layout_passes=False),
        scratch_types=[
            pltpu.VMEM((LANES,), jnp.int32),
            pltpu.VMEM((TB,), jnp.int32),
            pltpu.VMEM((TB + LANES,), jnp.int32),
            pltpu.VMEM((TB + LANES,), jnp.float32),
            pltpu.VMEM((TB, N_DIM), jnp.float32),
            pltpu.VMEM((SEG_CHUNK, N_DIM), jnp.float32),
            pltpu.SemaphoreType.DMA,
            pltpu.SemaphoreType.DMA,
            pltpu.SemaphoreType.DMA,
            pltpu.SemaphoreType.DMA,
        ],
    )
    return f(bounds, relids, segids, rval, rel_emb)


def _s3_body(bounds, adj0, adj1, ev, feats, vhat, ha, out, pe_out,
             bnd_v, seg_v, e_v, idx_v, f_rows, v_rows, dot_buf, dots_v,
             acc_v, ha_v, pe_st, sem, sem2, sem3, sem4, sem5):
    w = _worker_id()
    pltpu.sync_copy(bounds.at[w], bnd_v)
    bv = bnd_v[...]
    b0 = bv[0]
    b1 = bv[1]
    n0 = w * NODE_CHUNK
    _zero_rows(acc_v, NODE_ACC)
    io = lax.iota(jnp.int32, LANES)
    pltpu.sync_copy(ha, ha_v)
    one16 = jnp.full((LANES,), 1.0, jnp.float32)

    def pinit(r, carry):
        pe_st[pl.ds(r * LANES, LANES)] = one16
        return carry
    lax.fori_loop(0, (NODE_ACC + LANES - 1) // LANES, pinit, None)
    lane0 = io == 0
    zv = jnp.zeros((LANES,), jnp.float32)

    a8 = (b0 // 8) * 8
    nbat = (b1 - a8 + TB - 1) // TB

    def batch(k, bcarry):
        d_cur = bcarry[0]
        sreg = bcarry[1]
        accs = bcarry[2:]
        base = a8 + k * TB
        c1 = pltpu.async_copy(adj1.at[pl.ds(base, TB)], idx_v, sem2)
        c2 = pltpu.async_copy(
            adj0.at[pl.ds(base, TB)], seg_v.at[pl.ds(0, TB)], sem3)
        c3 = pltpu.async_copy(ev.at[pl.ds(base, TB)], e_v.at[pl.ds(0, TB)], sem4)
        c4 = pltpu.async_copy(vhat.at[pl.ds(base, TB)], v_rows, sem5)
        c1.wait()
        cg = pltpu.async_copy(feats.at[idx_v], f_rows, sem)
        c2.wait()
        c3.wait()
        c4.wait()
        cg.wait()
        lo = jnp.maximum(b0 - base, 0)
        hi = jnp.minimum(b1 - base, TB)

        def dgrp(g, dcarry):
            for u in range(LANES):
                i = g * LANES + u
                p0 = f_rows[i, pl.ds(0, LANES)] * v_rows[i, pl.ds(0, LANES)]
                p1 = f_rows[i, pl.ds(LANES, LANES)] * v_rows[i, pl.ds(LANES, LANES)]
                for c in range(2, DSL, 2):
                    sl = pl.ds(c * LANES, LANES)
                    sl2 = pl.ds((c + 1) * LANES, LANES)
                    p0 = p0 + f_rows[i, sl] * v_rows[i, sl]
                    p1 = p1 + f_rows[i, sl2] * v_rows[i, sl2]
                dot_buf[u, pl.ds(0, LANES)] = p0 + p1
            cols = [plsc.load_gather(dot_buf, [io, jnp.full((LANES,), c, jnp.int32)])
                    for c in range(LANES)]
            while len(cols) > 1:
                cols = [cols[t] + cols[t + 1] for t in range(0, len(cols), 2)]
            dots_v[pl.ds(g * LANES, LANES)] = cols[0]
            return dcarry
        lax.fori_loop(0, TB // LANES, dgrp, None)

        def edge(i, ecarry):
            dc = ecarry[0]
            sr = ecarry[1]
            ac = ecarry[2:]
            d_i = seg_v[pl.ds(i, LANES)][0] - n0
            ibc = jnp.full((LANES,), i, jnp.int32)
            eeb = plsc.load_gather(e_v, [ibc])
            dotb = plsc.load_gather(dots_v, [ibc])
            coefb = dotb * (-2.0 * eeb)
            flush = d_i != dc

            def _do_flush(dc_, sr_, ac_):
                rsv = 1.0 / (sr_ + 1e-16)
                hacc = None
                for c in range(DSL):
                    sl = pl.ds(c * LANES, LANES)
                    t = ac_[c] * rsv
                    fvec = 1.0 - 2.0 / (jnp.exp(t + t) + 1.0)
                    acc_v[dc_, sl] = fvec
                    hp = fvec * ha_v[sl]
                    hacc = hp if hacc is None else hacc + hp
                for kk in (8, 4, 2, 1):
                    dot_buf[0, pl.ds(0, LANES)] = hacc
                    hacc = hacc + plsc.load_gather(
                        dot_buf, [jnp.zeros((LANES,), jnp.int32), io ^ kk])
                plsc.store_scatter(
                    pe_st, [jnp.full((LANES,), dc_, jnp.int32)],
                    jnp.exp(hacc), mask=lane0)

            @pl.when(flush & (dc >= 0))
            def _():
                _do_flush(dc, sr, ac)

            new_ac = []
            for c in range(DSL):
                sl = pl.ds(c * LANES, LANES)
                contrib = eeb * f_rows[i, sl] + coefb * v_rows[i, sl]
                new_ac.append(jnp.where(flush, contrib, ac[c] + contrib))
            sr_new = jnp.where(flush, eeb, sr + eeb)
            return (d_i, sr_new) + tuple(new_ac)
        ecarry = lax.fori_loop(lo, hi, edge, (d_cur, sreg) + accs)
        return ecarry
    fcarry = lax.fori_loop(
        0, nbat, batch, (-1, zv) + (zv,) * DSL)
    d_cur = fcarry[0]
    sreg = fcarry[1]
    accs = fcarry[2:]

    @pl.when(d_cur >= 0)
    def _():
        rsv = 1.0 / (sreg + 1e-16)
        hacc = None
        for c in range(DSL):
            sl = pl.ds(c * LANES, LANES)
            t = accs[c] * rsv
            fvec = 1.0 - 2.0 / (jnp.exp(t + t) + 1.0)
            acc_v[d_cur, sl] = fvec
            hp = fvec * ha_v[sl]
            hacc = hp if hacc is None else hacc + hp
        for kk in (8, 4, 2, 1):
            dot_buf[0, pl.ds(0, LANES)] = hacc
            hacc = hacc + plsc.load_gather(
                dot_buf, [jnp.zeros((LANES,), jnp.int32), io ^ kk])
        plsc.store_scatter(
            pe_st, [jnp.full((LANES,), d_cur, jnp.int32)],
            jnp.exp(hacc), mask=lane0)

    pltpu.sync_copy(acc_v.at[pl.ds(0, NODE_CHUNK)], out.at[pl.ds(n0, NODE_CHUNK)])
    pltpu.sync_copy(pe_st.at[pl.ds(0, NODE_CHUNK)], pe_out.at[pl.ds(n0, NODE_CHUNK)])

    @pl.when(w == NW - 1)
    def _():
        pltpu.sync_copy(
            acc_v.at[pl.ds(NODE_CHUNK, NODE_LAST - NODE_CHUNK)],
            out.at[pl.ds(NW * NODE_CHUNK, NODE_LAST - NODE_CHUNK)])
        pltpu.sync_copy(
            pe_st.at[pl.ds(NODE_CHUNK, NODE_LAST - NODE_CHUNK)],
            pe_out.at[pl.ds(NW * NODE_CHUNK, NODE_LAST - NODE_CHUNK)])


def _s3_call(bounds, adj0, adj1, ev, feats, vhat, ha):
    f = pl.kernel(
        _s3_body,
        out_type=[jax.ShapeDtypeStruct((N_NODE, N_DIM), jnp.float32),
                  jax.ShapeDtypeStruct((N_NODE,), jnp.float32)],
        mesh=_mesh_(),
        compiler_params=pltpu.CompilerParams(needs_layout_passes=False),
        scratch_types=[
            pltpu.VMEM((LANES,), jnp.int32),
            pltpu.VMEM((TB + LANES,), jnp.int32),
            pltpu.VMEM((TB + LANES,), jnp.float32),
            pltpu.VMEM((TB,), jnp.int32),
            pltpu.VMEM((TB, N_DIM), jnp.float32),
            pltpu.VMEM((TB, N_DIM), jnp.float32),
            pltpu.VMEM((LANES, LANES), jnp.float32),
            pltpu.VMEM((TB + LANES,), jnp.float32),
            pltpu.VMEM((NODE_ACC, N_DIM), jnp.float32),
            pltpu.VMEM((N_DIM,), jnp.float32),
            pltpu.VMEM((NODE_ACC + LANES,), jnp.float32),
            pltpu.SemaphoreType.DMA,
            pltpu.SemaphoreType.DMA,
            pltpu.SemaphoreType.DMA,
            pltpu.SemaphoreType.DMA,
            pltpu.SemaphoreType.DMA,
        ],
    )
    return f(bounds, adj0, adj1, ev, feats, vhat, ha)


def _s4_body(bounds, hi0, hi1, pe, feats, ha, out, pe_out,
             bnd_v, seg_v, idx_v, pe_v, q_buf, f_rows, acc_v, ha_v, pe_st,
             dot_scr, sem, sem2, sem3):
    w = _worker_id()
    pltpu.sync_copy(bounds.at[w], bnd_v)
    bv = bnd_v[...]
    b0 = bv[0]
    b1 = bv[1]
    n0 = w * NODE_CHUNK
    pltpu.sync_copy(pe, pe_v.at[pl.ds(0, N_NODE)])
    _zero_rows(acc_v, NODE_ACC)
    io = lax.iota(jnp.int32, LANES)
    pltpu.sync_copy(ha, ha_v)
    one16 = jnp.full((LANES,), 1.0, jnp.float32)

    def pinit(r, carry):
        pe_st[pl.ds(r * LANES, LANES)] = one16
        return carry
    lax.fori_loop(0, (NODE_ACC + LANES - 1) // LANES, pinit, None)
    lane0 = io == 0

    a8 = (b0 // 8) * 8
    nbat = (b1 - a8 + TB - 1) // TB

    zv = jnp.zeros((LANES,), jnp.float32)

    def batch(k, bcarry):
        d_cur = bcarry[0]
        sreg = bcarry[1]
        accs = bcarry[2:]
        base = a8 + k * TB
        c1 = pltpu.async_copy(hi1.at[pl.ds(base, TB)], idx_v, sem2)
        c2 = pltpu.async_copy(
            hi0.at[pl.ds(base, TB)], seg_v.at[pl.ds(0, TB)], sem3)
        c1.wait()
        cg = pltpu.async_copy(feats.at[idx_v], f_rows, sem)

        def qg(g, qcarry):
            sv = idx_v[pl.ds(g * LANES, LANES)]
            q_buf[pl.ds(g * LANES, LANES)] = plsc.load_gather(pe_v, [sv])
            return qcarry
        lax.fori_loop(0, TB // LANES, qg, None)
        c2.wait()
        cg.wait()
        lo = jnp.maximum(b0 - base, 0)
        hi = jnp.minimum(b1 - base, TB)

        def edge(i, ecarry):
            dc = ecarry[0]
            sr = ecarry[1]
            ac = ecarry[2:]
            d_i = seg_v[pl.ds(i, LANES)][0] - n0
            qb = plsc.load_gather(q_buf, [jnp.full((LANES,), i, jnp.int32)])
            flush = d_i != dc

            @pl.when(flush & (dc >= 0))
            def _():
                rsv = 1.0 / (sr + 1e-16)
                hacc = None
                for c in range(DSL):
                    sl = pl.ds(c * LANES, LANES)
                    t = ac[c] * rsv
                    fvec = 1.0 - 2.0 / (jnp.exp(t + t) + 1.0)
                    acc_v[dc, sl] = fvec
                    hp = fvec * ha_v[sl]
                    hacc = hp if hacc is None else hacc + hp
                for kk in (8, 4, 2, 1):
                    dot_scr[...] = hacc
                    hacc = hacc + plsc.load_gather(dot_scr, [io ^ kk])
                plsc.store_scatter(
                    pe_st, [jnp.full((LANES,), dc, jnp.int32)],
                    jnp.exp(hacc), mask=lane0)

            new_ac = []
            for c in range(DSL):
                sl = pl.ds(c * LANES, LANES)
                contrib = qb * f_rows[i, sl]
                new_ac.append(jnp.where(flush, contrib, ac[c] + contrib))
            sr_new = jnp.where(flush, qb, sr + qb)
            return (d_i, sr_new) + tuple(new_ac)
        ecarry = lax.fori_loop(lo, hi, edge, (d_cur, sreg) + accs)
        return ecarry
    fcarry = lax.fori_loop(0, nbat, batch, (-1, zv) + (zv,) * DSL)
    d_cur = fcarry[0]
    sreg = fcarry[1]
    accs = fcarry[2:]

    @pl.when(d_cur >= 0)
    def _():
        rsv = 1.0 / (sreg + 1e-16)
        hacc = None
        for c in range(DSL):
            sl = pl.ds(c * LANES, LANES)
            t = accs[c] * rsv
            fvec = 1.0 - 2.0 / (jnp.exp(t + t) + 1.0)
            acc_v[d_cur, sl] = fvec
            hp = fvec * ha_v[sl]
            hacc = hp if hacc is None else hacc + hp
        for kk in (8, 4, 2, 1):
            dot_scr[...] = hacc
            hacc = hacc + plsc.load_gather(dot_scr, [io ^ kk])
        plsc.store_scatter(
            pe_st, [jnp.full((LANES,), d_cur, jnp.int32)],
            jnp.exp(hacc), mask=lane0)

    pltpu.sync_copy(acc_v.at[pl.ds(0, NODE_CHUNK)], out.at[pl.ds(n0, NODE_CHUNK)])
    pltpu.sync_copy(pe_st.at[pl.ds(0, NODE_CHUNK)], pe_out.at[pl.ds(n0, NODE_CHUNK)])

    @pl.when(w == NW - 1)
    def _():
        pltpu.sync_copy(
            acc_v.at[pl.ds(NODE_CHUNK, NODE_LAST - NODE_CHUNK)],
            out.at[pl.ds(NW * NODE_CHUNK, NODE_LAST - NODE_CHUNK)])
        pltpu.sync_copy(
            pe_st.at[pl.ds(NODE_CHUNK, NODE_LAST - NODE_CHUNK)],
            pe_out.at[pl.ds(NW * NODE_CHUNK, NODE_LAST - NODE_CHUNK)])


def _s4_call(bounds, hi0, hi1, pe, feats, ha):
    f = pl.kernel(
        _s4_body,
        out_type=[jax.ShapeDtypeStruct((N_NODE, N_DIM), jnp.float32),
                  jax.ShapeDtypeStruct((N_NODE,), jnp.float32)],
        mesh=_mesh_(),
        compiler_params=pltpu.CompilerParams(needs_layout_passes=False),
        scratch_types=[
            pltpu.VMEM((LANES,), jnp.int32),
            pltpu.VMEM((TB + LANES,), jnp.int32),
            pltpu.VMEM((TB,), jnp.int32),
            pltpu.VMEM((N_NODE + LANES,), jnp.float32),
            pltpu.VMEM((TB + LANES,), jnp.float32),
            pltpu.VMEM((TB, N_DIM), jnp.float32),
            pltpu.VMEM((NODE_ACC, N_DIM), jnp.float32),
            pltpu.VMEM((N_DIM,), jnp.float32),
            pltpu.VMEM((NODE_ACC + LANES,), jnp.float32),
            pltpu.VMEM((LANES,), jnp.float32),
            pltpu.SemaphoreType.DMA,
            pltpu.SemaphoreType.DMA,
            pltpu.SemaphoreType.DMA,
        ],
    )
    return f(bounds, hi0, hi1, pe, feats, ha)


def _tanh_body(x_ref, o_ref):
    o_ref[...] = jnp.tanh(x_ref[...])


def _tanh_call(x):
    return pl.pallas_call(
        _tanh_body,
        out_shape=jax.ShapeDtypeStruct(x.shape, jnp.float32),
    )(x)


def _t1_body(tri_ref, k01_ref, v_ref, e_ref):
    x = tri_ref[...]
    ss = jnp.sum(x * x, axis=1, keepdims=True)
    v = x / jnp.maximum(jnp.sqrt(ss), 1e-12)
    v_ref[...] = v
    e_ref[...] = jnp.exp(v @ k01_ref[...])


def _t1_call(tri, k01):
    n = tri.shape[0]
    br = 1024
    return pl.pallas_call(
        _t1_body,
        grid=(n // br,),
        in_specs=[pl.BlockSpec((br, N_DIM), lambda i: (i, 0)),
                  pl.BlockSpec((N_DIM, 2), lambda i: (0, 0))],
        out_specs=[pl.BlockSpec((br, N_DIM), lambda i: (i, 0)),
                   pl.BlockSpec((br, 2), lambda i: (i, 0))],
        out_shape=[jax.ShapeDtypeStruct((n, N_DIM), jnp.float32),
                   jax.ShapeDtypeStruct((n, 2), jnp.float32)],
    )(tri, k01)


def kernel(features, rel_emb, adj, r_index, r_val, high_nei,
           attn_k0, attn_k1, high_att0, high_att1):
    r0, r1 = r_index[0], r_index[1]
    a0, a1 = adj[0], adj[1]
    h0, h1 = high_nei[0], high_nei[1]

    seg_grid = jnp.arange(N_SEG_CHUNKS + 1, dtype=jnp.int32) * SEG_CHUNK
    sb = jnp.searchsorted(r0, seg_grid, side="left").astype(jnp.int32)
    s1_bounds = jnp.zeros((NW * SEG_CHUNKS_PER_W, LANES), jnp.int32)
    s1_bounds = s1_bounds.at[:N_SEG_CHUNKS, 0].set(sb[:-1])
    s1_bounds = s1_bounds.at[:N_SEG_CHUNKS, 1].set(sb[1:])

    node_grid = jnp.concatenate([
        jnp.arange(NW, dtype=jnp.int32) * NODE_CHUNK,
        jnp.array([N_NODE], jnp.int32)])
    ab = jnp.searchsorted(a0, node_grid, side="left").astype(jnp.int32)
    s3_bounds = jnp.zeros((NW, LANES), jnp.int32)
    s3_bounds = s3_bounds.at[:, 0].set(ab[:-1]).at[:, 1].set(ab[1:])
    hb = jnp.searchsorted(h0, node_grid, side="left").astype(jnp.int32)
    s4_bounds = jnp.zeros((NW, LANES), jnp.int32)
    s4_bounds = s4_bounds.at[:, 0].set(hb[:-1]).at[:, 1].set(hb[1:])

    pad1 = lambda x: jnp.pad(x, (0, PAD))
    r0p, r1p, rvp = pad1(r0), pad1(r1), pad1(r_val)
    a0p, a1p = pad1(a0), pad1(a1)
    h0p, h1p = pad1(h0), pad1(h1)

    tri = _s1_call(s1_bounds, r1p, r0p, rvp, rel_emb)
    k01 = jnp.concatenate([attn_k0, attn_k1], axis=1)
    vhat, e01 = _t1_call(tri, k01)
    e0 = jnp.asarray(e01[:, 0])
    e1 = jnp.asarray(e01[:, 1])

    ha0 = jnp.squeeze(high_att0, -1)
    ha1 = jnp.squeeze(high_att1, -1)
    f0 = _tanh_call(features)
    f1, _ = _s3_call(s3_bounds, a0p, a1p, e0, f0, vhat, ha0)
    f2, pe0 = _s3_call(s3_bounds, a0p, a1p, e1, f1, vhat, ha0)
    f3, pe1 = _s4_call(s4_bounds, h0p, h1p, pe0, f2, ha1)
    f4, _ = _s4_call(s4_bounds, h0p, h1p, pe1, f3, ha1)
    return jnp.concatenate([f0, f1, f2, f3, f4], axis=-1)
